# baseline (speedup 1.0000x reference)
import functools

import jax
import jax.numpy as jnp
import numpy as np
from jax import lax
from jax.experimental import pallas as pl
from jax.experimental.pallas import tpu as pltpu
from jax.experimental.pallas import tpu_sc as plsc

L = 16
NT = 16
FPAD = 16
S_INNER = 9
INT_MIN = np.int32(-2147483648)


def _ffnn_body(x_ref, w1_ref, b1_ref, w2_ref, b2_ref, w3_ref, o_ref):
    x = x_ref[...]
    h = jnp.dot(x, w1_ref[...], preferred_element_type=jnp.float32)
    h = jnp.maximum(h + b1_ref[0:1, :], 0.0)
    h = jnp.dot(h, w2_ref[...], preferred_element_type=jnp.float32)
    h = jnp.maximum(h + b2_ref[0:1, :], 0.0)
    o_ref[...] = jnp.dot(h, w3_ref[...], preferred_element_type=jnp.float32)


def _ffnn_scores(g_i, W1, b1, W2, b2, W3):
    n, d_in = g_i.shape
    hid = W1.shape[1]
    HP = 256
    W1p = jnp.zeros((d_in, HP), jnp.float32).at[:, :hid].set(W1)
    b1p = jnp.zeros((8, HP), jnp.float32).at[0, :hid].set(b1)
    W2p = jnp.zeros((HP, HP), jnp.float32).at[:hid, :hid].set(W2)
    b2p = jnp.zeros((8, HP), jnp.float32).at[0, :hid].set(b2)
    W3p = jnp.zeros((HP, 128), jnp.float32).at[:hid, 0].set(W3[:, 0])
    BM = 2000
    assert n % BM == 0
    out = pl.pallas_call(
        _ffnn_body,
        grid=(n // BM,),
        in_specs=[
            pl.BlockSpec((BM, d_in), lambda i: (i, 0)),
            pl.BlockSpec((d_in, HP), lambda i: (0, 0)),
            pl.BlockSpec((8, HP), lambda i: (0, 0)),
            pl.BlockSpec((HP, HP), lambda i: (0, 0)),
            pl.BlockSpec((8, HP), lambda i: (0, 0)),
            pl.BlockSpec((HP, 128), lambda i: (0, 0)),
        ],
        out_specs=pl.BlockSpec((BM, 128), lambda i: (i, 0)),
        out_shape=jax.ShapeDtypeStruct((n, 128), jnp.float32),
    )(g_i, W1p, b1p, W2p, b2p, W3p)
    return out[:, 0]


def _make_sc_prune(n_pad, n_real):
    assert n_pad % (NT * L) == 0
    PT = n_pad // NT
    NG = PT // L
    NW = n_pad // 32
    WPT = NW // NT
    assert WPT % 8 == 0
    NGGLOB = n_pad // L
    iota = lambda: lax.iota(jnp.int32, L)

    mesh = plsc.VectorSubcoreMesh(
        core_axis_name="c", subcore_axis_name="s",
        num_cores=1, num_subcores=NT)

    @functools.partial(
        pl.kernel,
        out_type=[
            jax.ShapeDtypeStruct((n_pad,), jnp.float32),
            jax.ShapeDtypeStruct((n_pad,), jnp.int32),
            jax.ShapeDtypeStruct((NW,), jnp.int32),
            jax.ShapeDtypeStruct((16,), jnp.int32),
        ],
        mesh=mesh,
        compiler_params=pltpu.CompilerParams(needs_layout_passes=False),
        scratch_types=[
            pltpu.VMEM((n_pad,), jnp.int32),
            pltpu.VMEM((n_pad,), jnp.int32),
            pltpu.VMEM((n_pad,), jnp.int32),
            pltpu.VMEM((n_pad,), jnp.float32),
            pltpu.VMEM((n_pad,), jnp.int32),
            pltpu.VMEM((NW,), jnp.int32),
            pltpu.VMEM((NW,), jnp.int32),
            pltpu.VMEM((NW,), jnp.int32),
            pltpu.VMEM((NGGLOB,), jnp.int32),
            pltpu.VMEM((NGGLOB,), jnp.int32),
            pltpu.VMEM((16,), jnp.int32),
            pltpu.VMEM((16,), jnp.int32),
            pltpu.VMEM((8 * NT,), jnp.int32),
            pltpu.VMEM((256 * L,), jnp.int32),
            pltpu.VMEM_SHARED((NW,), jnp.int32),
            pltpu.VMEM_SHARED((8 * NT,), jnp.int32),
        ],
    )
    def prune(sm_hbm, st_hbm, ln_hbm, k_hbm, bits_hbm,
              out_s_hbm, out_m_hbm, bits_out_hbm, flag_hbm,
              pk_v, aux_v, key_v, sm_v, wlim_v, bits_v, prev_v, mapfl_v,
              grng_v, dirty_v, kv_v, fb_v, fl_v, hist_v, sh_bits, sh_flags):
        wid = lax.axis_index("s") + lax.axis_index("c") * NT

        pltpu.sync_copy(sm_hbm, sm_v)
        pltpu.sync_copy(st_hbm, pk_v)
        pltpu.sync_copy(ln_hbm, aux_v)
        pltpu.sync_copy(k_hbm, kv_v)
        pltpu.sync_copy(bits_hbm, bits_v)
        kth = kv_v[...][0]

        def rd_word(ref, wd):
            wb = (wd >> 4) << 4
            v = ref[pl.ds(wb, L)]
            return jnp.sum(jnp.where(iota() == wd - wb, v, 0))

        def init_g(g, _):
            sl = pl.ds(g * L, L)
            st = pk_v[sl]
            ln = aux_v[sl]
            pk_v[sl] = (st << 4) | ln
            b = lax.bitcast_convert_type(sm_v[sl], jnp.int32)
            key = jnp.where(b >= 0, b, b ^ np.int32(0x7FFFFFFF))
            ivec = g * L + iota()
            valid = (ivec >= FPAD) & (ivec < FPAD + n_real)
            key_v[sl] = jnp.where(valid, key, INT_MIN)
            return 0

        lax.fori_loop(0, NGGLOB, init_g, 0)

        def srch(g, _):
            base = wid * PT + g * L
            ivec = base + iota()
            sl = pl.ds(base, L)
            pk_i = pk_v[sl]
            s_i = jnp.right_shift(pk_i, 4)
            e_i = s_i + (pk_i & 15)
            t_lo = s_i - 9

            def bstep(p, pos, leq, tgt):
                step = jnp.left_shift(np.int32(1), 14 - p)
                cand = pos + step
                jg = jnp.clip(cand - 1, 0, n_pad - 1)
                s_c = jnp.right_shift(plsc.load_gather(pk_v, [jg]), 4)
                less = (s_c <= tgt) if leq else (s_c < tgt)
                ok = (cand <= n_pad) & less
                return jnp.where(ok, cand, pos)

            def lo_loop(p, pos):
                return bstep(p, pos, False, t_lo)

            def hi_loop(p, pos):
                return bstep(p, pos, True, e_i)

            lo = lax.fori_loop(0, 15, lo_loop, jnp.zeros((L,), jnp.int32))
            hi = lax.fori_loop(0, 15, hi_loop, jnp.zeros((L,), jnp.int32))
            dl = ivec - lo
            dr = hi - ivec - 1
            wlim_v[sl] = (dl << 16) | dr
            lo_w = jnp.clip(jnp.right_shift(base - jnp.max(dl), 5), 0, NW - 1)
            hi_w = jnp.clip(jnp.right_shift(base + 15 + jnp.max(dr), 5),
                            0, NW - 1)
            gg = wid * NG + g
            wb = (gg >> 4) << 4
            lane = gg - wb
            blk = grng_v[pl.ds(wb, L)]
            grng_v[pl.ds(wb, L)] = jnp.where(
                iota() == lane, (lo_w << 16) | hi_w, blk)
            return 0

        lax.fori_loop(0, NG, srch, 0)

        def init_pd(mw, _):
            sl = pl.ds(mw * L, L)
            prev_v[sl] = bits_v[sl]
            return 0

        lax.fori_loop(0, NW // L, init_pd, 0)

        def init_d(gi, _):
            dirty_v[pl.ds(wid * NG + gi * L, L)] = jnp.full((L,), 1, jnp.int32)
            return 0

        lax.fori_loop(0, NG // L, init_d, 0)

        def kp_bits(jc):
            w = plsc.load_gather(bits_v, [jnp.right_shift(jc, 5)])
            return jnp.right_shift(w, jc & 31) & 1

        def sweep(s, prev):
            fb_v[...] = jnp.zeros((L,), jnp.int32)

            @pl.when(prev != 0)
            def _do_sweep():
                def group(g, _):
                    gg = wid * NG + g
                    dirt = rd_word(dirty_v, gg)

                    @pl.when(dirt != 0)
                    def _scan():
                        base = wid * PT + g * L
                        ivec = base + iota()
                        sl = pl.ds(base, L)
                        pk_i = pk_v[sl]
                        s_i = jnp.right_shift(pk_i, 4)
                        e_i = s_i + (pk_i & 15)
                        key_i = key_v[sl]
                        wl = wlim_v[sl]
                        dl = jnp.right_shift(wl, 16)
                        dr = wl & 65535
                        val_i = (ivec >= FPAD) & (ivec < FPAD + n_real)

                        def win_body(is_left, d, thr):
                            j = ivec - d if is_left else ivec + d
                            jc = jnp.clip(j, 0, n_pad - 1)
                            pk_j = plsc.load_gather(pk_v, [jc])
                            s_j = jnp.right_shift(pk_j, 4)
                            e_j = s_j + (pk_j & 15)
                            key_j = plsc.load_gather(key_v, [jc])
                            kp = kp_bits(jc)
                            if is_left:
                                inw = d <= dl
                                hit = inw & (s_j < s_i) & (s_i <= e_j) & \
                                    (e_j < e_i) & (key_j >= key_i) & (kp == 1)
                            else:
                                inw = d <= dr
                                hit = inw & (s_j > s_i) & (e_j > e_i) & \
                                    (key_j > key_i) & (kp == 1)
                            return thr | jnp.where(hit, 1, 0)

                        z16 = jnp.zeros((L,), jnp.int32)
                        thr = plsc.parallel_loop(
                            np.int32(1), jnp.max(dl) + 1, carry=z16)(
                                functools.partial(win_body, True))
                        thr = plsc.parallel_loop(
                            np.int32(1), jnp.max(dr) + 1, carry=thr)(
                                functools.partial(win_body, False))
                        new_keep = jnp.where((thr == 0) & val_i, 1, 0)
                        hw = jnp.sum(new_keep << iota())
                        wd = jnp.right_shift(gg, 1)
                        sh = (gg & 1) * 16
                        wb = (wd >> 4) << 4
                        lane = wd - wb
                        blk = bits_v[pl.ds(wb, L)]
                        old = jnp.sum(jnp.where(iota() == lane, blk, 0))
                        neww = (old & ~(65535 << sh)) | (hw << sh)
                        bits_v[pl.ds(wb, L)] = jnp.where(
                            iota() == lane, neww, blk)
                        ch = jnp.where(neww != old, 1, 0)
                        fb_v[...] = fb_v[...] | jnp.full((L,), ch, jnp.int32)

                    return 0

                lax.fori_loop(0, NG, group, 0)

            pltpu.sync_copy(bits_v.at[pl.ds(wid * WPT, WPT)],
                            sh_bits.at[pl.ds(wid * WPT, WPT)])
            pltpu.sync_copy(fb_v.at[pl.ds(0, 8)],
                            sh_flags.at[pl.ds(wid * 8, 8)])
            plsc.subcore_barrier()
            pltpu.sync_copy(sh_bits, bits_v)
            pltpu.sync_copy(sh_flags, fl_v)
            plsc.subcore_barrier()

            @pl.when(prev != 0)
            def _mark_dirty():
                def bld(mw, _):
                    msl = pl.ds(mw * L, L)
                    nv = bits_v[msl]
                    mapfl_v[msl] = jnp.where(nv != prev_v[msl], 1, 0)
                    prev_v[msl] = nv
                    return 0

                lax.fori_loop(0, NW // L, bld, 0)

                def mkd(gi, _):
                    gsl = pl.ds(wid * NG + gi * L, L)
                    rng = grng_v[gsl]
                    lo_w = jnp.right_shift(rng, 16)
                    span = (rng & 65535) - lo_w
                    d = jnp.where(span > 7, 1, 0)
                    for t in range(8):
                        f = plsc.load_gather(
                            mapfl_v, [jnp.clip(lo_w + t, 0, NW - 1)])
                        d = d | jnp.where((t <= span) & (f == 1), 1, 0)
                    dirty_v[gsl] = d
                    return 0

                lax.fori_loop(0, NG // L, mkd, 0)

            def orf(t, a):
                return a | fl_v[pl.ds(t * L, L)]

            vacc = lax.fori_loop(0, 8 * NT // L, orf,
                                 jnp.zeros((L,), jnp.int32))
            return jnp.any(vacc != 0).astype(jnp.int32)

        not_conv = lax.fori_loop(0, S_INNER, sweep, np.int32(1))

        pltpu.sync_copy(bits_v.at[pl.ds(wid * WPT, WPT)],
                        bits_out_hbm.at[pl.ds(wid * WPT, WPT)])
        fb_v[...] = jnp.full((L,), not_conv, jnp.int32)

        @pl.when(wid == 0)
        def _wflag():
            pltpu.sync_copy(fb_v, flag_hbm)

        def unpack16(gg):
            wd = jnp.right_shift(gg, 1)
            sh = (gg & 1) * 16
            w = rd_word(bits_v, wd)
            return jnp.right_shift(w, sh + iota()) & 1

        def cnt_g(g, acc):
            kp = unpack16(wid * NG + g)
            return acc + jnp.sum(kp)

        my_cnt = lax.fori_loop(0, NG, cnt_g, np.int32(0))
        fb_v[...] = jnp.full((L,), my_cnt, jnp.int32)
        pltpu.sync_copy(fb_v.at[pl.ds(0, 8)], sh_flags.at[pl.ds(wid * 8, 8)])
        plsc.subcore_barrier()
        pltpu.sync_copy(sh_flags, fl_v)
        plsc.subcore_barrier()

        def sumf(t, a):
            return a + fl_v[pl.ds(t * L, L)]

        vsum = lax.fori_loop(0, 8 * NT // L, sumf, jnp.zeros((L,), jnp.int32))
        total = jnp.right_shift(jnp.sum(vsum), 3)

        fb_v[...] = jnp.where(iota() < 8, INT_MIN, 0)

        @pl.when(total > kth)
        def _select():
            def level(p, carry):
                rem, hi = carry
                shift = 24 - 8 * p

                def zero_h(w, _):
                    hist_v[pl.ds(w * L, L)] = jnp.zeros((L,), jnp.int32)
                    return 0

                lax.fori_loop(0, 256, zero_h, 0)

                def acc_g(g, _):
                    sl = pl.ds(g * L, L)
                    key = key_v[sl]
                    kp = unpack16(g)
                    pref_ok = jnp.where(
                        p == 0,
                        jnp.ones((L,), jnp.bool_),
                        (key >> jnp.minimum(shift + 8, 31)) == hi)
                    cand = jnp.where((kp == 1) & pref_ok, 1, 0)
                    bn = jnp.where(p == 0, (key >> 24) + 128,
                                   (key >> shift) & 255)
                    plsc.addupdate_scatter(hist_v, [bn * L + iota()], cand)
                    return 0

                lax.fori_loop(0, NGGLOB, acc_g, 0)

                def scan_b(br, st):
                    b = 255 - br
                    found, bstar, acc, rem_n = st
                    hb = jnp.sum(hist_v[pl.ds(b * L, L)])
                    acc2 = acc + hb
                    take = (found == 0) & (acc2 >= rem)
                    bstar = jnp.where(take, b, bstar)
                    rem_n = jnp.where(take, rem - acc, rem_n)
                    found = jnp.where(take, 1, found)
                    return (found, bstar, acc2, rem_n)

                _, bstar, _, rem_n = lax.fori_loop(
                    0, 256, scan_b,
                    (np.int32(0), np.int32(0), np.int32(0), rem))
                bval = jnp.where(p == 0, bstar - 128, bstar)
                return (rem_n, (hi << 8) | bval)

            rem, hi = lax.fori_loop(0, 4, level, (kth, np.int32(0)))
            vstar = hi
            fb_v[...] = jnp.where(iota() < 8, vstar, rem)

            def rank_g(g, c):
                sl = pl.ds(g * L, L)
                key = key_v[sl]
                kp = unpack16(g)
                tie = jnp.where((kp == 1) & (key == vstar), 1, 0)
                incl = jnp.cumsum(tie)
                pk_v[sl] = c + incl - tie
                return c + jnp.sum(tie)

            lax.fori_loop(0, NGGLOB, rank_g, np.int32(0))

        fbv = fb_v[...]
        vstar = fbv[0]
        mquota = fbv[8]

        def out_g(g, _):
            gg = wid * NG + g
            base = gg * L
            sl = pl.ds(base, L)
            kp = unpack16(gg)
            key = key_v[sl]
            rank = pk_v[sl]
            fin = (kp == 1) & ((key > vstar) |
                               ((key == vstar) & (rank < mquota)))
            aux_v[sl] = jnp.where(fin, 1, 0)
            sm_v[sl] = jnp.where(fin, sm_v[sl], 0.0)
            return 0

        lax.fori_loop(0, NG, out_g, 0)
        pltpu.sync_copy(sm_v.at[pl.ds(wid * PT, PT)],
                        out_s_hbm.at[pl.ds(wid * PT, PT)])
        pltpu.sync_copy(aux_v.at[pl.ds(wid * PT, PT)],
                        out_m_hbm.at[pl.ds(wid * PT, PT)])

    return prune


def kernel(g_i, W1, b1, W2, b2, W3, b3, span_start, span_len, T):
    n = g_i.shape[0]
    s_m = _ffnn_scores(g_i, W1, b1, W2, b2, W3) + b3[0]

    n_pad = ((FPAD + n + 4095) // 4096) * 4096
    back = n_pad - FPAD - n
    st32 = span_start.astype(jnp.int32)
    ln32 = span_len.astype(jnp.int32)
    st_p = jnp.concatenate([
        jnp.zeros((FPAD,), jnp.int32), st32,
        50000 + lax.iota(jnp.int32, back)])
    ln_p = jnp.concatenate([
        jnp.zeros((FPAD,), jnp.int32), ln32, jnp.zeros((back,), jnp.int32)])
    sm_p = jnp.concatenate([
        jnp.zeros((FPAD,), jnp.float32), s_m, jnp.zeros((back,), jnp.float32)])
    k = (0.4 * jnp.asarray(T).astype(jnp.float32)).astype(jnp.int32)
    k_arr = jnp.full((16,), k, jnp.int32)

    nw = n_pad // 32
    widx = jnp.arange(nw, dtype=jnp.int32)
    w_last = (FPAD + n) // 32
    rem_bits = (FPAD + n) % 32
    last_val = (1 << rem_bits) - 1 if rem_bits else 0
    bits0 = jnp.where(widx == 0, np.int32(-65536), np.int32(-1))
    bits0 = jnp.where(widx == w_last, np.int32(last_val), bits0)
    bits0 = jnp.where(widx > w_last, 0, bits0)

    prune = _make_sc_prune(n_pad, n)

    def cond(carry):
        return carry[3] != 0

    def body(carry):
        bits, _, _, _ = carry
        out_s, out_m, bits2, flag = prune(sm_p, st_p, ln_p, k_arr, bits)
        return (bits2, out_s, out_m, flag[0])

    _, out_s, out_m, _ = lax.while_loop(
        cond, body,
        (bits0, jnp.zeros((n_pad,), jnp.float32),
         jnp.zeros((n_pad,), jnp.int32), np.int32(1)))
    pruned = out_s[FPAD:FPAD + n]
    mask = out_m[FPAD:FPAD + n].astype(bool)
    return pruned, mask


# speedup vs baseline: 1.0080x; 1.0080x over previous
import functools

import jax
import jax.numpy as jnp
import numpy as np
from jax import lax
from jax.experimental import pallas as pl
from jax.experimental.pallas import tpu as pltpu
from jax.experimental.pallas import tpu_sc as plsc

L = 16
NT = 16
S_INNER = 9
INT_MIN = np.int32(-2147483648)


def _ffnn_body(x_ref, w1_ref, b1_ref, w2_ref, b2_ref, w3_ref, o_ref):
    x = x_ref[...]
    h = jnp.dot(x, w1_ref[...], preferred_element_type=jnp.float32)
    h = jnp.maximum(h + b1_ref[0:1, :], 0.0)
    h = jnp.dot(h, w2_ref[...], preferred_element_type=jnp.float32)
    h = jnp.maximum(h + b2_ref[0:1, :], 0.0)
    o_ref[...] = jnp.dot(h, w3_ref[...], preferred_element_type=jnp.float32)


def _ffnn_scores(g_i, W1, b1, W2, b2, W3):
    n, d_in = g_i.shape
    hid = W1.shape[1]
    HP = 256
    W1p = jnp.zeros((d_in, HP), jnp.float32).at[:, :hid].set(W1)
    b1p = jnp.zeros((8, HP), jnp.float32).at[0, :hid].set(b1)
    W2p = jnp.zeros((HP, HP), jnp.float32).at[:hid, :hid].set(W2)
    b2p = jnp.zeros((8, HP), jnp.float32).at[0, :hid].set(b2)
    W3p = jnp.zeros((HP, 8), jnp.float32).at[:hid, 0].set(W3[:, 0])
    BM = 2000
    assert n % BM == 0
    out = pl.pallas_call(
        _ffnn_body,
        grid=(n // BM,),
        in_specs=[
            pl.BlockSpec((BM, d_in), lambda i: (i, 0)),
            pl.BlockSpec((d_in, HP), lambda i: (0, 0)),
            pl.BlockSpec((8, HP), lambda i: (0, 0)),
            pl.BlockSpec((HP, HP), lambda i: (0, 0)),
            pl.BlockSpec((8, HP), lambda i: (0, 0)),
            pl.BlockSpec((HP, 8), lambda i: (0, 0)),
        ],
        out_specs=pl.BlockSpec((BM, 8), lambda i: (i, 0)),
        out_shape=jax.ShapeDtypeStruct((n, 8), jnp.float32),
    )(g_i, W1p, b1p, W2p, b2p, W3p)
    return out[:, 0]


def _make_sc_prune(n_pad, n_real):
    assert n_pad % (NT * L) == 0
    PT = n_pad // NT
    NG = PT // L
    NW = n_pad // 32
    WPT = NW // NT
    assert WPT % 8 == 0
    NGGLOB = n_pad // L
    iota = lambda: lax.iota(jnp.int32, L)

    mesh = plsc.VectorSubcoreMesh(
        core_axis_name="c", subcore_axis_name="s",
        num_cores=1, num_subcores=NT)

    @functools.partial(
        pl.kernel,
        out_type=[
            jax.ShapeDtypeStruct((n_pad,), jnp.float32),
            jax.ShapeDtypeStruct((n_pad,), jnp.int32),
            jax.ShapeDtypeStruct((NW,), jnp.int32),
            jax.ShapeDtypeStruct((16,), jnp.int32),
        ],
        mesh=mesh,
        compiler_params=pltpu.CompilerParams(needs_layout_passes=False),
        scratch_types=[
            pltpu.VMEM((n_pad,), jnp.int32),
            pltpu.VMEM((n_pad,), jnp.int32),
            pltpu.VMEM((n_pad,), jnp.int32),
            pltpu.VMEM((n_pad,), jnp.float32),
            pltpu.VMEM((n_pad,), jnp.int32),
            pltpu.VMEM((NW,), jnp.int32),
            pltpu.VMEM((NW,), jnp.int32),
            pltpu.VMEM((NW,), jnp.int32),
            pltpu.VMEM((NGGLOB,), jnp.int32),
            pltpu.VMEM((NGGLOB,), jnp.int32),
            pltpu.VMEM((16,), jnp.int32),
            pltpu.VMEM((16,), jnp.int32),
            pltpu.VMEM((8 * NT,), jnp.int32),
            pltpu.VMEM((256 * L,), jnp.int32),
            pltpu.VMEM_SHARED((NW,), jnp.int32),
            pltpu.VMEM_SHARED((8 * NT,), jnp.int32),
        ],
    )
    def prune(sm_hbm, st_hbm, ln_hbm, k_hbm, bits_hbm,
              out_s_hbm, out_m_hbm, bits_out_hbm, flag_hbm,
              pk_v, aux_v, key_v, sm_v, wlim_v, bits_v, prev_v, mapfl_v,
              grng_v, dirty_v, kv_v, fb_v, fl_v, hist_v, sh_bits, sh_flags):
        wid = lax.axis_index("s") + lax.axis_index("c") * NT

        pltpu.sync_copy(sm_hbm, sm_v.at[pl.ds(0, n_real)])
        pltpu.sync_copy(st_hbm, pk_v.at[pl.ds(0, n_real)])
        pltpu.sync_copy(ln_hbm, aux_v.at[pl.ds(0, n_real)])
        pltpu.sync_copy(k_hbm, kv_v)
        pltpu.sync_copy(bits_hbm, bits_v)
        kth = kv_v[...][0]

        def rd_word(ref, wd):
            wb = (wd >> 4) << 4
            v = ref[pl.ds(wb, L)]
            return jnp.sum(jnp.where(iota() == wd - wb, v, 0))

        def init_g(g, _):
            sl = pl.ds(g * L, L)
            ivec = g * L + iota()
            valid = ivec < n_real
            st = jnp.where(valid, pk_v[sl], 50000 + ivec)
            ln = jnp.where(valid, aux_v[sl], 0)
            pk_v[sl] = (st << 4) | ln
            b = lax.bitcast_convert_type(sm_v[sl], jnp.int32)
            key = jnp.where(b >= 0, b, b ^ np.int32(0x7FFFFFFF))
            key_v[sl] = jnp.where(valid, key, INT_MIN)
            return 0

        lax.fori_loop(0, NGGLOB, init_g, 0)

        def srch(g, _):
            base = wid * PT + g * L
            ivec = base + iota()
            sl = pl.ds(base, L)
            pk_i = pk_v[sl]
            s_i = jnp.right_shift(pk_i, 4)
            e_i = s_i + (pk_i & 15)
            t_lo = s_i - 9

            def bstep(p, pos, leq, tgt):
                step = jnp.left_shift(np.int32(1), 14 - p)
                cand = pos + step
                jg = jnp.clip(cand - 1, 0, n_pad - 1)
                s_c = jnp.right_shift(plsc.load_gather(pk_v, [jg]), 4)
                less = (s_c <= tgt) if leq else (s_c < tgt)
                ok = (cand <= n_pad) & less
                return jnp.where(ok, cand, pos)

            def lo_loop(p, pos):
                return bstep(p, pos, False, t_lo)

            def hi_loop(p, pos):
                return bstep(p, pos, True, e_i)

            lo = lax.fori_loop(0, 15, lo_loop, jnp.zeros((L,), jnp.int32))
            hi = lax.fori_loop(0, 15, hi_loop, jnp.zeros((L,), jnp.int32))
            dl = ivec - lo
            dr = hi - ivec - 1
            wlim_v[sl] = (dl << 16) | dr
            lo_w = jnp.clip(jnp.right_shift(base - jnp.max(dl), 5), 0, NW - 1)
            hi_w = jnp.clip(jnp.right_shift(base + 15 + jnp.max(dr), 5),
                            0, NW - 1)
            gg = wid * NG + g
            wb = (gg >> 4) << 4
            lane = gg - wb
            blk = grng_v[pl.ds(wb, L)]
            grng_v[pl.ds(wb, L)] = jnp.where(
                iota() == lane, (lo_w << 16) | hi_w, blk)
            return 0

        lax.fori_loop(0, NG, srch, 0)

        def init_pd(mw, _):
            sl = pl.ds(mw * L, L)
            prev_v[sl] = bits_v[sl]
            return 0

        lax.fori_loop(0, NW // L, init_pd, 0)

        def init_d(gi, _):
            dirty_v[pl.ds(wid * NG + gi * L, L)] = jnp.full((L,), 1, jnp.int32)
            return 0

        lax.fori_loop(0, NG // L, init_d, 0)

        def kp_bits(jc):
            w = plsc.load_gather(bits_v, [jnp.right_shift(jc, 5)])
            return jnp.right_shift(w, jc & 31) & 1

        def sweep(s, prev):
            fb_v[...] = jnp.zeros((L,), jnp.int32)

            @pl.when(prev != 0)
            def _do_sweep():
                def group(g, _):
                    gg = wid * NG + g
                    dirt = rd_word(dirty_v, gg)

                    @pl.when(dirt != 0)
                    def _scan():
                        base = wid * PT + g * L
                        ivec = base + iota()
                        sl = pl.ds(base, L)
                        pk_i = pk_v[sl]
                        s_i = jnp.right_shift(pk_i, 4)
                        e_i = s_i + (pk_i & 15)
                        key_i = key_v[sl]
                        wl = wlim_v[sl]
                        dl = jnp.right_shift(wl, 16)
                        dr = wl & 65535
                        val_i = ivec < n_real

                        def win_body(is_left, d, thr):
                            j = ivec - d if is_left else ivec + d
                            jc = jnp.clip(j, 0, n_pad - 1)
                            pk_j = plsc.load_gather(pk_v, [jc])
                            s_j = jnp.right_shift(pk_j, 4)
                            e_j = s_j + (pk_j & 15)
                            key_j = plsc.load_gather(key_v, [jc])
                            kp = kp_bits(jc)
                            if is_left:
                                inw = d <= dl
                                hit = inw & (s_j < s_i) & (s_i <= e_j) & \
                                    (e_j < e_i) & (key_j >= key_i) & (kp == 1)
                            else:
                                inw = d <= dr
                                hit = inw & (s_j > s_i) & (e_j > e_i) & \
                                    (key_j > key_i) & (kp == 1)
                            return thr | jnp.where(hit, 1, 0)

                        z16 = jnp.zeros((L,), jnp.int32)
                        thr = plsc.parallel_loop(
                            np.int32(1), jnp.max(dl) + 1, carry=z16)(
                                functools.partial(win_body, True))
                        thr = plsc.parallel_loop(
                            np.int32(1), jnp.max(dr) + 1, carry=thr)(
                                functools.partial(win_body, False))
                        new_keep = jnp.where((thr == 0) & val_i, 1, 0)
                        hw = jnp.sum(new_keep << iota())
                        wd = jnp.right_shift(gg, 1)
                        sh = (gg & 1) * 16
                        wb = (wd >> 4) << 4
                        lane = wd - wb
                        blk = bits_v[pl.ds(wb, L)]
                        old = jnp.sum(jnp.where(iota() == lane, blk, 0))
                        neww = (old & ~(65535 << sh)) | (hw << sh)
                        bits_v[pl.ds(wb, L)] = jnp.where(
                            iota() == lane, neww, blk)
                        ch = jnp.where(neww != old, 1, 0)
                        fb_v[...] = fb_v[...] | jnp.full((L,), ch, jnp.int32)

                    return 0

                lax.fori_loop(0, NG, group, 0)

            pltpu.sync_copy(bits_v.at[pl.ds(wid * WPT, WPT)],
                            sh_bits.at[pl.ds(wid * WPT, WPT)])
            pltpu.sync_copy(fb_v.at[pl.ds(0, 8)],
                            sh_flags.at[pl.ds(wid * 8, 8)])
            plsc.subcore_barrier()
            pltpu.sync_copy(sh_bits, bits_v)
            pltpu.sync_copy(sh_flags, fl_v)
            plsc.subcore_barrier()

            @pl.when(prev != 0)
            def _mark_dirty():
                def bld(mw, _):
                    msl = pl.ds(mw * L, L)
                    nv = bits_v[msl]
                    mapfl_v[msl] = jnp.where(nv != prev_v[msl], 1, 0)
                    prev_v[msl] = nv
                    return 0

                lax.fori_loop(0, NW // L, bld, 0)

                def mkd(gi, _):
                    gsl = pl.ds(wid * NG + gi * L, L)
                    rng = grng_v[gsl]
                    lo_w = jnp.right_shift(rng, 16)
                    span = (rng & 65535) - lo_w
                    d = jnp.where(span > 7, 1, 0)
                    for t in range(8):
                        f = plsc.load_gather(
                            mapfl_v, [jnp.clip(lo_w + t, 0, NW - 1)])
                        d = d | jnp.where((t <= span) & (f == 1), 1, 0)
                    dirty_v[gsl] = d
                    return 0

                lax.fori_loop(0, NG // L, mkd, 0)

            def orf(t, a):
                return a | fl_v[pl.ds(t * L, L)]

            vacc = lax.fori_loop(0, 8 * NT // L, orf,
                                 jnp.zeros((L,), jnp.int32))
            return jnp.any(vacc != 0).astype(jnp.int32)

        not_conv = lax.fori_loop(0, S_INNER, sweep, np.int32(1))

        pltpu.sync_copy(bits_v.at[pl.ds(wid * WPT, WPT)],
                        bits_out_hbm.at[pl.ds(wid * WPT, WPT)])
        fb_v[...] = jnp.full((L,), not_conv, jnp.int32)

        @pl.when(wid == 0)
        def _wflag():
            pltpu.sync_copy(fb_v, flag_hbm)

        def unpack16(gg):
            wd = jnp.right_shift(gg, 1)
            sh = (gg & 1) * 16
            w = rd_word(bits_v, wd)
            return jnp.right_shift(w, sh + iota()) & 1

        def cnt_g(g, acc):
            kp = unpack16(wid * NG + g)
            return acc + jnp.sum(kp)

        my_cnt = lax.fori_loop(0, NG, cnt_g, np.int32(0))
        fb_v[...] = jnp.full((L,), my_cnt, jnp.int32)
        pltpu.sync_copy(fb_v.at[pl.ds(0, 8)], sh_flags.at[pl.ds(wid * 8, 8)])
        plsc.subcore_barrier()
        pltpu.sync_copy(sh_flags, fl_v)
        plsc.subcore_barrier()

        def sumf(t, a):
            return a + fl_v[pl.ds(t * L, L)]

        vsum = lax.fori_loop(0, 8 * NT // L, sumf, jnp.zeros((L,), jnp.int32))
        total = jnp.right_shift(jnp.sum(vsum), 3)

        fb_v[...] = jnp.where(iota() < 8, INT_MIN, 0)

        @pl.when(total > kth)
        def _select():
            def level(p, carry):
                rem, hi = carry
                shift = 24 - 8 * p

                def zero_h(w, _):
                    hist_v[pl.ds(w * L, L)] = jnp.zeros((L,), jnp.int32)
                    return 0

                lax.fori_loop(0, 256, zero_h, 0)

                def acc_g(g, _):
                    sl = pl.ds(g * L, L)
                    key = key_v[sl]
                    kp = unpack16(g)
                    pref_ok = jnp.where(
                        p == 0,
                        jnp.ones((L,), jnp.bool_),
                        (key >> jnp.minimum(shift + 8, 31)) == hi)
                    cand = jnp.where((kp == 1) & pref_ok, 1, 0)
                    bn = jnp.where(p == 0, (key >> 24) + 128,
                                   (key >> shift) & 255)
                    plsc.addupdate_scatter(hist_v, [bn * L + iota()], cand)
                    return 0

                lax.fori_loop(0, NGGLOB, acc_g, 0)

                def scan_b(br, st):
                    b = 255 - br
                    found, bstar, acc, rem_n = st
                    hb = jnp.sum(hist_v[pl.ds(b * L, L)])
                    acc2 = acc + hb
                    take = (found == 0) & (acc2 >= rem)
                    bstar = jnp.where(take, b, bstar)
                    rem_n = jnp.where(take, rem - acc, rem_n)
                    found = jnp.where(take, 1, found)
                    return (found, bstar, acc2, rem_n)

                _, bstar, _, rem_n = lax.fori_loop(
                    0, 256, scan_b,
                    (np.int32(0), np.int32(0), np.int32(0), rem))
                bval = jnp.where(p == 0, bstar - 128, bstar)
                return (rem_n, (hi << 8) | bval)

            rem, hi = lax.fori_loop(0, 4, level, (kth, np.int32(0)))
            vstar = hi
            fb_v[...] = jnp.where(iota() < 8, vstar, rem)

            def rank_g(g, c):
                sl = pl.ds(g * L, L)
                key = key_v[sl]
                kp = unpack16(g)
                tie = jnp.where((kp == 1) & (key == vstar), 1, 0)
                incl = jnp.cumsum(tie)
                pk_v[sl] = c + incl - tie
                return c + jnp.sum(tie)

            lax.fori_loop(0, NGGLOB, rank_g, np.int32(0))

        fbv = fb_v[...]
        vstar = fbv[0]
        mquota = fbv[8]

        def out_g(g, _):
            gg = wid * NG + g
            base = gg * L
            sl = pl.ds(base, L)
            kp = unpack16(gg)
            key = key_v[sl]
            rank = pk_v[sl]
            fin = (kp == 1) & ((key > vstar) |
                               ((key == vstar) & (rank < mquota)))
            aux_v[sl] = jnp.where(fin, 1, 0)
            sm_v[sl] = jnp.where(fin, sm_v[sl], 0.0)
            return 0

        lax.fori_loop(0, NG, out_g, 0)
        pltpu.sync_copy(sm_v.at[pl.ds(wid * PT, PT)],
                        out_s_hbm.at[pl.ds(wid * PT, PT)])
        pltpu.sync_copy(aux_v.at[pl.ds(wid * PT, PT)],
                        out_m_hbm.at[pl.ds(wid * PT, PT)])

    return prune


def kernel(g_i, W1, b1, W2, b2, W3, b3, span_start, span_len, T):
    n = g_i.shape[0]
    s_m = _ffnn_scores(g_i, W1, b1, W2, b2, W3) + b3[0]

    n_pad = ((n + 4095) // 4096) * 4096
    st32 = span_start.astype(jnp.int32)
    ln32 = span_len.astype(jnp.int32)
    k = (0.4 * jnp.asarray(T).astype(jnp.float32)).astype(jnp.int32)
    k_arr = jnp.full((16,), k, jnp.int32)

    nw = n_pad // 32
    widx = jnp.arange(nw, dtype=jnp.int32)
    w_last = n // 32
    rem_bits = n % 32
    last_val = (1 << rem_bits) - 1 if rem_bits else 0
    bits0 = jnp.where(widx < w_last, np.int32(-1), np.int32(0))
    bits0 = jnp.where(widx == w_last, np.int32(last_val), bits0)

    prune = _make_sc_prune(n_pad, n)

    def cond(carry):
        return carry[3] != 0

    def body(carry):
        bits, _, _, _ = carry
        out_s, out_m, bits2, flag = prune(s_m, st32, ln32, k_arr, bits)
        return (bits2, out_s, out_m, flag[0])

    _, out_s, out_m, _ = lax.while_loop(
        cond, body,
        (bits0, jnp.zeros((n_pad,), jnp.float32),
         jnp.zeros((n_pad,), jnp.int32), np.int32(1)))
    pruned = out_s[:n]
    mask = out_m[:n].astype(bool)
    return pruned, mask


# speedup vs baseline: 1.0504x; 1.0421x over previous
import functools

import jax
import jax.numpy as jnp
import numpy as np
from jax import lax
from jax.experimental import pallas as pl
from jax.experimental.pallas import tpu as pltpu
from jax.experimental.pallas import tpu_sc as plsc

L = 16
NT = 16
S_INNER = 9
INT_MIN = np.int32(-2147483648)


def _ffnn_body(x_ref, w1_ref, b1_ref, w2_ref, b2_ref, w3_ref, o_ref):
    x = x_ref[...]
    h = jnp.dot(x, w1_ref[...], preferred_element_type=jnp.float32)
    h = jnp.maximum(h + b1_ref[0:1, :], 0.0)
    h = jnp.dot(h, w2_ref[...], preferred_element_type=jnp.float32)
    h = jnp.maximum(h + b2_ref[0:1, :], 0.0)
    o_ref[...] = jnp.dot(h, w3_ref[...], preferred_element_type=jnp.float32)


def _ffnn_scores(g_i, W1, b1, W2, b2, W3):
    n, d_in = g_i.shape
    hid = W1.shape[1]
    HP = 256
    W1p = jnp.zeros((d_in, HP), jnp.float32).at[:, :hid].set(W1)
    b1p = jnp.zeros((8, HP), jnp.float32).at[0, :hid].set(b1)
    W2p = jnp.zeros((HP, HP), jnp.float32).at[:hid, :hid].set(W2)
    b2p = jnp.zeros((8, HP), jnp.float32).at[0, :hid].set(b2)
    W3p = jnp.zeros((HP, 8), jnp.float32).at[:hid, 0].set(W3[:, 0])
    BM = 2000
    assert n % BM == 0
    out = pl.pallas_call(
        _ffnn_body,
        grid=(n // BM,),
        in_specs=[
            pl.BlockSpec((BM, d_in), lambda i: (i, 0)),
            pl.BlockSpec((d_in, HP), lambda i: (0, 0)),
            pl.BlockSpec((8, HP), lambda i: (0, 0)),
            pl.BlockSpec((HP, HP), lambda i: (0, 0)),
            pl.BlockSpec((8, HP), lambda i: (0, 0)),
            pl.BlockSpec((HP, 8), lambda i: (0, 0)),
        ],
        out_specs=pl.BlockSpec((BM, 8), lambda i: (i, 0)),
        out_shape=jax.ShapeDtypeStruct((n, 8), jnp.float32),
    )(g_i, W1p, b1p, W2p, b2p, W3p)
    return out[:, 0]


def _make_sc_prune(n_pad, n_real):
    assert n_pad % (NT * L) == 0
    PT = n_pad // NT
    NG = PT // L
    NW = n_pad // 32
    WPT = NW // NT
    assert WPT % 8 == 0
    NGGLOB = n_pad // L
    iota = lambda: lax.iota(jnp.int32, L)

    mesh = plsc.VectorSubcoreMesh(
        core_axis_name="c", subcore_axis_name="s",
        num_cores=1, num_subcores=NT)

    @functools.partial(
        pl.kernel,
        out_type=[
            jax.ShapeDtypeStruct((n_pad,), jnp.float32),
            jax.ShapeDtypeStruct((n_pad,), jnp.int32),
            jax.ShapeDtypeStruct((NW,), jnp.int32),
            jax.ShapeDtypeStruct((16,), jnp.int32),
        ],
        mesh=mesh,
        compiler_params=pltpu.CompilerParams(needs_layout_passes=False),
        scratch_types=[
            pltpu.VMEM((n_pad,), jnp.int32),
            pltpu.VMEM((n_pad,), jnp.int32),
            pltpu.VMEM((n_pad,), jnp.int32),
            pltpu.VMEM((n_pad,), jnp.float32),
            pltpu.VMEM((n_pad,), jnp.int32),
            pltpu.VMEM((NW,), jnp.int32),
            pltpu.VMEM((NW,), jnp.int32),
            pltpu.VMEM((NW,), jnp.int32),
            pltpu.VMEM((NGGLOB,), jnp.int32),
            pltpu.VMEM((NGGLOB,), jnp.int32),
            pltpu.VMEM((16,), jnp.int32),
            pltpu.VMEM((16,), jnp.int32),
            pltpu.VMEM((8 * NT,), jnp.int32),
            pltpu.VMEM((256 * L,), jnp.int32),
            pltpu.VMEM_SHARED((NW,), jnp.int32),
            pltpu.VMEM_SHARED((8 * NT,), jnp.int32),
            pltpu.VMEM_SHARED((n_pad,), jnp.int32),
            pltpu.VMEM_SHARED((n_pad,), jnp.int32),
        ],
    )
    def prune(sm_hbm, st_hbm, ln_hbm, k_hbm, bits_hbm,
              out_s_hbm, out_m_hbm, bits_out_hbm, flag_hbm,
              pk_v, aux_v, key_v, sm_v, wlim_v, bits_v, prev_v, mapfl_v,
              grng_v, dirty_v, kv_v, fb_v, fl_v, hist_v, sh_bits, sh_flags,
              sh_pk, sh_key):
        wid = lax.axis_index("s") + lax.axis_index("c") * NT

        pltpu.sync_copy(sm_hbm, sm_v.at[pl.ds(0, n_real)])
        pltpu.sync_copy(st_hbm, pk_v.at[pl.ds(0, n_real)])
        pltpu.sync_copy(ln_hbm, aux_v.at[pl.ds(0, n_real)])
        pltpu.sync_copy(k_hbm, kv_v)
        pltpu.sync_copy(bits_hbm, bits_v)
        kth = kv_v[...][0]

        def rd_word(ref, wd):
            wb = (wd >> 4) << 4
            v = ref[pl.ds(wb, L)]
            return jnp.sum(jnp.where(iota() == wd - wb, v, 0))

        def init_g(g, _):
            gg = wid * NG + g
            sl = pl.ds(gg * L, L)
            ivec = gg * L + iota()
            valid = ivec < n_real
            st = jnp.where(valid, pk_v[sl], 50000 + ivec)
            ln = jnp.where(valid, aux_v[sl], 0)
            pk_v[sl] = (st << 4) | ln
            b = lax.bitcast_convert_type(sm_v[sl], jnp.int32)
            key = jnp.where(b >= 0, b, b ^ np.int32(0x7FFFFFFF))
            key_v[sl] = jnp.where(valid, key, INT_MIN)
            return 0

        lax.fori_loop(0, NG, init_g, 0)
        pltpu.sync_copy(pk_v.at[pl.ds(wid * PT, PT)],
                        sh_pk.at[pl.ds(wid * PT, PT)])
        pltpu.sync_copy(key_v.at[pl.ds(wid * PT, PT)],
                        sh_key.at[pl.ds(wid * PT, PT)])
        plsc.subcore_barrier()
        pltpu.sync_copy(sh_pk, pk_v)
        pltpu.sync_copy(sh_key, key_v)
        plsc.subcore_barrier()

        def srch(g, _):
            base = wid * PT + g * L
            ivec = base + iota()
            sl = pl.ds(base, L)
            pk_i = pk_v[sl]
            s_i = jnp.right_shift(pk_i, 4)
            e_i = s_i + (pk_i & 15)
            t_lo = s_i - 9

            def bstep(p, pos, leq, tgt):
                step = jnp.left_shift(np.int32(1), 14 - p)
                cand = pos + step
                jg = jnp.clip(cand - 1, 0, n_pad - 1)
                s_c = jnp.right_shift(plsc.load_gather(pk_v, [jg]), 4)
                less = (s_c <= tgt) if leq else (s_c < tgt)
                ok = (cand <= n_pad) & less
                return jnp.where(ok, cand, pos)

            def lo_loop(p, pos):
                return bstep(p, pos, False, t_lo)

            def hi_loop(p, pos):
                return bstep(p, pos, True, e_i)

            lo = lax.fori_loop(0, 15, lo_loop, jnp.zeros((L,), jnp.int32))
            hi = lax.fori_loop(0, 15, hi_loop, jnp.zeros((L,), jnp.int32))
            dl = ivec - lo
            dr = hi - ivec - 1
            wlim_v[sl] = (dl << 16) | dr
            lo_w = jnp.clip(jnp.right_shift(base - jnp.max(dl), 5), 0, NW - 1)
            hi_w = jnp.clip(jnp.right_shift(base + 15 + jnp.max(dr), 5),
                            0, NW - 1)
            gg = wid * NG + g
            wb = (gg >> 4) << 4
            lane = gg - wb
            blk = grng_v[pl.ds(wb, L)]
            grng_v[pl.ds(wb, L)] = jnp.where(
                iota() == lane, (lo_w << 16) | hi_w, blk)
            return 0

        lax.fori_loop(0, NG, srch, 0)

        def init_pd(mw, _):
            sl = pl.ds(mw * L, L)
            prev_v[sl] = bits_v[sl]
            return 0

        lax.fori_loop(0, NW // L, init_pd, 0)

        def init_d(gi, _):
            dirty_v[pl.ds(wid * NG + gi * L, L)] = jnp.full((L,), 1, jnp.int32)
            return 0

        lax.fori_loop(0, NG // L, init_d, 0)

        def kp_bits(jc):
            w = plsc.load_gather(bits_v, [jnp.right_shift(jc, 5)])
            return jnp.right_shift(w, jc & 31) & 1

        def sweep(s, prev):
            fb_v[...] = jnp.zeros((L,), jnp.int32)

            @pl.when(prev != 0)
            def _do_sweep():
                def group(g, _):
                    gg = wid * NG + g
                    dirt = rd_word(dirty_v, gg)

                    @pl.when(dirt != 0)
                    def _scan():
                        base = wid * PT + g * L
                        ivec = base + iota()
                        sl = pl.ds(base, L)
                        pk_i = pk_v[sl]
                        s_i = jnp.right_shift(pk_i, 4)
                        e_i = s_i + (pk_i & 15)
                        key_i = key_v[sl]
                        wl = wlim_v[sl]
                        dl = jnp.right_shift(wl, 16)
                        dr = wl & 65535
                        val_i = ivec < n_real

                        def win_body(is_left, d, thr):
                            j = ivec - d if is_left else ivec + d
                            jc = jnp.clip(j, 0, n_pad - 1)
                            pk_j = plsc.load_gather(pk_v, [jc])
                            s_j = jnp.right_shift(pk_j, 4)
                            e_j = s_j + (pk_j & 15)
                            key_j = plsc.load_gather(key_v, [jc])
                            kp = kp_bits(jc)
                            if is_left:
                                inw = d <= dl
                                hit = inw & (s_j < s_i) & (s_i <= e_j) & \
                                    (e_j < e_i) & (key_j >= key_i) & (kp == 1)
                            else:
                                inw = d <= dr
                                hit = inw & (s_j > s_i) & (e_j > e_i) & \
                                    (key_j > key_i) & (kp == 1)
                            return thr | jnp.where(hit, 1, 0)

                        z16 = jnp.zeros((L,), jnp.int32)
                        thr = plsc.parallel_loop(
                            np.int32(1), jnp.max(dl) + 1, unroll=2,
                            carry=z16)(functools.partial(win_body, True))
                        thr = plsc.parallel_loop(
                            np.int32(1), jnp.max(dr) + 1, unroll=2,
                            carry=thr)(functools.partial(win_body, False))
                        new_keep = jnp.where((thr == 0) & val_i, 1, 0)
                        hw = jnp.sum(new_keep << iota())
                        wd = jnp.right_shift(gg, 1)
                        sh = (gg & 1) * 16
                        wb = (wd >> 4) << 4
                        lane = wd - wb
                        blk = bits_v[pl.ds(wb, L)]
                        old = jnp.sum(jnp.where(iota() == lane, blk, 0))
                        neww = (old & ~(65535 << sh)) | (hw << sh)
                        bits_v[pl.ds(wb, L)] = jnp.where(
                            iota() == lane, neww, blk)
                        ch = jnp.where(neww != old, 1, 0)
                        fb_v[...] = fb_v[...] | jnp.full((L,), ch, jnp.int32)

                    return 0

                lax.fori_loop(0, NG, group, 0)

            pltpu.sync_copy(bits_v.at[pl.ds(wid * WPT, WPT)],
                            sh_bits.at[pl.ds(wid * WPT, WPT)])
            pltpu.sync_copy(fb_v.at[pl.ds(0, 8)],
                            sh_flags.at[pl.ds(wid * 8, 8)])
            plsc.subcore_barrier()
            pltpu.sync_copy(sh_bits, bits_v)
            pltpu.sync_copy(sh_flags, fl_v)
            plsc.subcore_barrier()

            @pl.when(prev != 0)
            def _mark_dirty():
                def bld(mw, _):
                    msl = pl.ds(mw * L, L)
                    nv = bits_v[msl]
                    mapfl_v[msl] = jnp.where(nv != prev_v[msl], 1, 0)
                    prev_v[msl] = nv
                    return 0

                lax.fori_loop(0, NW // L, bld, 0)

                def mkd(gi, _):
                    gsl = pl.ds(wid * NG + gi * L, L)
                    rng = grng_v[gsl]
                    lo_w = jnp.right_shift(rng, 16)
                    span = (rng & 65535) - lo_w
                    d = jnp.where(span > 7, 1, 0)
                    for t in range(8):
                        f = plsc.load_gather(
                            mapfl_v, [jnp.clip(lo_w + t, 0, NW - 1)])
                        d = d | jnp.where((t <= span) & (f == 1), 1, 0)
                    dirty_v[gsl] = d
                    return 0

                lax.fori_loop(0, NG // L, mkd, 0)

            def orf(t, a):
                return a | fl_v[pl.ds(t * L, L)]

            vacc = lax.fori_loop(0, 8 * NT // L, orf,
                                 jnp.zeros((L,), jnp.int32))
            return jnp.any(vacc != 0).astype(jnp.int32)

        not_conv = lax.fori_loop(0, S_INNER, sweep, np.int32(1))

        pltpu.sync_copy(bits_v.at[pl.ds(wid * WPT, WPT)],
                        bits_out_hbm.at[pl.ds(wid * WPT, WPT)])
        fb_v[...] = jnp.full((L,), not_conv, jnp.int32)

        @pl.when(wid == 0)
        def _wflag():
            pltpu.sync_copy(fb_v, flag_hbm)

        def unpack16(gg):
            wd = jnp.right_shift(gg, 1)
            sh = (gg & 1) * 16
            w = rd_word(bits_v, wd)
            return jnp.right_shift(w, sh + iota()) & 1

        def cnt_g(g, acc):
            kp = unpack16(wid * NG + g)
            return acc + jnp.sum(kp)

        my_cnt = lax.fori_loop(0, NG, cnt_g, np.int32(0))
        fb_v[...] = jnp.full((L,), my_cnt, jnp.int32)
        pltpu.sync_copy(fb_v.at[pl.ds(0, 8)], sh_flags.at[pl.ds(wid * 8, 8)])
        plsc.subcore_barrier()
        pltpu.sync_copy(sh_flags, fl_v)
        plsc.subcore_barrier()

        def sumf(t, a):
            return a + fl_v[pl.ds(t * L, L)]

        vsum = lax.fori_loop(0, 8 * NT // L, sumf, jnp.zeros((L,), jnp.int32))
        total = jnp.right_shift(jnp.sum(vsum), 3)

        fb_v[...] = jnp.where(iota() < 8, INT_MIN, 0)

        @pl.when(total > kth)
        def _select():
            def level(p, carry):
                rem, hi = carry
                shift = 24 - 8 * p

                def zero_h(w, _):
                    hist_v[pl.ds(w * L, L)] = jnp.zeros((L,), jnp.int32)
                    return 0

                lax.fori_loop(0, 256, zero_h, 0)

                def acc_g(g, _):
                    sl = pl.ds(g * L, L)
                    key = key_v[sl]
                    kp = unpack16(g)
                    pref_ok = jnp.where(
                        p == 0,
                        jnp.ones((L,), jnp.bool_),
                        (key >> jnp.minimum(shift + 8, 31)) == hi)
                    cand = jnp.where((kp == 1) & pref_ok, 1, 0)
                    bn = jnp.where(p == 0, (key >> 24) + 128,
                                   (key >> shift) & 255)
                    plsc.addupdate_scatter(hist_v, [bn * L + iota()], cand)
                    return 0

                lax.fori_loop(0, NGGLOB, acc_g, 0)

                def scan_b(br, st):
                    b = 255 - br
                    found, bstar, acc, rem_n = st
                    hb = jnp.sum(hist_v[pl.ds(b * L, L)])
                    acc2 = acc + hb
                    take = (found == 0) & (acc2 >= rem)
                    bstar = jnp.where(take, b, bstar)
                    rem_n = jnp.where(take, rem - acc, rem_n)
                    found = jnp.where(take, 1, found)
                    return (found, bstar, acc2, rem_n)

                _, bstar, _, rem_n = lax.fori_loop(
                    0, 256, scan_b,
                    (np.int32(0), np.int32(0), np.int32(0), rem))
                bval = jnp.where(p == 0, bstar - 128, bstar)
                return (rem_n, (hi << 8) | bval)

            rem, hi = lax.fori_loop(0, 4, level, (kth, np.int32(0)))
            vstar = hi
            fb_v[...] = jnp.where(iota() < 8, vstar, rem)

            def rank_g(g, c):
                sl = pl.ds(g * L, L)
                key = key_v[sl]
                kp = unpack16(g)
                tie = jnp.where((kp == 1) & (key == vstar), 1, 0)
                incl = jnp.cumsum(tie)
                pk_v[sl] = c + incl - tie
                return c + jnp.sum(tie)

            lax.fori_loop(0, NGGLOB, rank_g, np.int32(0))

        fbv = fb_v[...]
        vstar = fbv[0]
        mquota = fbv[8]

        def out_g(g, _):
            gg = wid * NG + g
            base = gg * L
            sl = pl.ds(base, L)
            kp = unpack16(gg)
            key = key_v[sl]
            rank = pk_v[sl]
            fin = (kp == 1) & ((key > vstar) |
                               ((key == vstar) & (rank < mquota)))
            aux_v[sl] = jnp.where(fin, 1, 0)
            sm_v[sl] = jnp.where(fin, sm_v[sl], 0.0)
            return 0

        lax.fori_loop(0, NG, out_g, 0)
        pltpu.sync_copy(sm_v.at[pl.ds(wid * PT, PT)],
                        out_s_hbm.at[pl.ds(wid * PT, PT)])
        pltpu.sync_copy(aux_v.at[pl.ds(wid * PT, PT)],
                        out_m_hbm.at[pl.ds(wid * PT, PT)])

    return prune


def kernel(g_i, W1, b1, W2, b2, W3, b3, span_start, span_len, T):
    n = g_i.shape[0]
    s_m = _ffnn_scores(g_i, W1, b1, W2, b2, W3) + b3[0]

    n_pad = ((n + 4095) // 4096) * 4096
    st32 = span_start.astype(jnp.int32)
    ln32 = span_len.astype(jnp.int32)
    k = (0.4 * jnp.asarray(T).astype(jnp.float32)).astype(jnp.int32)
    k_arr = jnp.full((16,), k, jnp.int32)

    nw = n_pad // 32
    widx = jnp.arange(nw, dtype=jnp.int32)
    w_last = n // 32
    rem_bits = n % 32
    last_val = (1 << rem_bits) - 1 if rem_bits else 0
    bits0 = jnp.where(widx < w_last, np.int32(-1), np.int32(0))
    bits0 = jnp.where(widx == w_last, np.int32(last_val), bits0)

    prune = _make_sc_prune(n_pad, n)

    def cond(carry):
        return carry[3] != 0

    def body(carry):
        bits, _, _, _ = carry
        out_s, out_m, bits2, flag = prune(s_m, st32, ln32, k_arr, bits)
        return (bits2, out_s, out_m, flag[0])

    _, out_s, out_m, _ = lax.while_loop(
        cond, body,
        (bits0, jnp.zeros((n_pad,), jnp.float32),
         jnp.zeros((n_pad,), jnp.int32), np.int32(1)))
    pruned = out_s[:n]
    mask = out_m[:n].astype(bool)
    return pruned, mask


# speedup vs baseline: 1.1067x; 1.0536x over previous
import functools

import jax
import jax.numpy as jnp
import numpy as np
from jax import lax
from jax.experimental import pallas as pl
from jax.experimental.pallas import tpu as pltpu
from jax.experimental.pallas import tpu_sc as plsc

L = 16
NT = 16
S_INNER = 9
INT_MIN = np.int32(-2147483648)


def _ffnn_body(x_ref, w1_ref, b1_ref, w2_ref, b2_ref, w3_ref, o_ref):
    x = x_ref[...]
    h = jnp.dot(x, w1_ref[...], preferred_element_type=jnp.float32)
    h = jnp.maximum(h + b1_ref[0:1, :], 0.0)
    h = jnp.dot(h, w2_ref[...], preferred_element_type=jnp.float32)
    h = jnp.maximum(h + b2_ref[0:1, :], 0.0)
    o_ref[...] = jnp.dot(h, w3_ref[...], preferred_element_type=jnp.float32)


def _ffnn_scores(g_i, W1, b1, W2, b2, W3):
    n, d_in = g_i.shape
    hid = W1.shape[1]
    HP = 256
    W1p = jnp.zeros((d_in, HP), jnp.float32).at[:, :hid].set(W1)
    b1p = jnp.zeros((8, HP), jnp.float32).at[0, :hid].set(b1)
    W2p = jnp.zeros((HP, HP), jnp.float32).at[:hid, :hid].set(W2)
    b2p = jnp.zeros((8, HP), jnp.float32).at[0, :hid].set(b2)
    W3p = jnp.zeros((HP, 8), jnp.float32).at[:hid, 0].set(W3[:, 0])
    BM = 2000
    assert n % BM == 0
    out = pl.pallas_call(
        _ffnn_body,
        grid=(n // BM,),
        in_specs=[
            pl.BlockSpec((BM, d_in), lambda i: (i, 0)),
            pl.BlockSpec((d_in, HP), lambda i: (0, 0)),
            pl.BlockSpec((8, HP), lambda i: (0, 0)),
            pl.BlockSpec((HP, HP), lambda i: (0, 0)),
            pl.BlockSpec((8, HP), lambda i: (0, 0)),
            pl.BlockSpec((HP, 8), lambda i: (0, 0)),
        ],
        out_specs=pl.BlockSpec((BM, 8), lambda i: (i, 0)),
        out_shape=jax.ShapeDtypeStruct((n, 8), jnp.float32),
    )(g_i, W1p, b1p, W2p, b2p, W3p)
    return out[:, 0]


def _make_sc_prune(n_pad, n_real):
    assert n_pad % (NT * L) == 0
    PT = n_pad // NT
    NG = PT // L
    NW = n_pad // 32
    WPT = NW // NT
    assert WPT % 8 == 0
    NGGLOB = n_pad // L
    iota = lambda: lax.iota(jnp.int32, L)

    mesh = plsc.VectorSubcoreMesh(
        core_axis_name="c", subcore_axis_name="s",
        num_cores=1, num_subcores=NT)

    @functools.partial(
        pl.kernel,
        out_type=[
            jax.ShapeDtypeStruct((n_pad,), jnp.float32),
            jax.ShapeDtypeStruct((n_pad,), jnp.int32),
            jax.ShapeDtypeStruct((NW,), jnp.int32),
            jax.ShapeDtypeStruct((16,), jnp.int32),
        ],
        mesh=mesh,
        compiler_params=pltpu.CompilerParams(needs_layout_passes=False),
        scratch_types=[
            pltpu.VMEM((n_pad,), jnp.int32),
            pltpu.VMEM((n_pad,), jnp.int32),
            pltpu.VMEM((n_pad,), jnp.int32),
            pltpu.VMEM((n_pad,), jnp.float32),
            pltpu.VMEM((n_pad,), jnp.int32),
            pltpu.VMEM((NW,), jnp.int32),
            pltpu.VMEM((NW,), jnp.int32),
            pltpu.VMEM((NW,), jnp.int32),
            pltpu.VMEM((NGGLOB,), jnp.int32),
            pltpu.VMEM((NGGLOB,), jnp.int32),
            pltpu.VMEM((16,), jnp.int32),
            pltpu.VMEM((16,), jnp.int32),
            pltpu.VMEM((8 * NT,), jnp.int32),
            pltpu.VMEM((256 * L,), jnp.int32),
            pltpu.VMEM_SHARED((NW,), jnp.int32),
            pltpu.VMEM_SHARED((8 * NT,), jnp.int32),
            pltpu.VMEM_SHARED((n_pad,), jnp.int32),
            pltpu.VMEM_SHARED((n_pad,), jnp.int32),
        ],
    )
    def prune(sm_hbm, st_hbm, ln_hbm, k_hbm, bits_hbm,
              out_s_hbm, out_m_hbm, bits_out_hbm, flag_hbm,
              pk_v, aux_v, key_v, sm_v, wlim_v, bits_v, prev_v, mapfl_v,
              grng_v, dirty_v, kv_v, fb_v, fl_v, hist_v, sh_bits, sh_flags,
              sh_pk, sh_key):
        wid = lax.axis_index("s") + lax.axis_index("c") * NT

        pltpu.sync_copy(sm_hbm, sm_v.at[pl.ds(0, n_real)])
        pltpu.sync_copy(st_hbm, pk_v.at[pl.ds(0, n_real)])
        pltpu.sync_copy(ln_hbm, aux_v.at[pl.ds(0, n_real)])
        pltpu.sync_copy(k_hbm, kv_v)
        pltpu.sync_copy(bits_hbm, bits_v)
        kth = kv_v[...][0]

        def rd_word(ref, wd):
            wb = (wd >> 4) << 4
            v = ref[pl.ds(wb, L)]
            return jnp.sum(jnp.where(iota() == wd - wb, v, 0))

        def init_g(g, _):
            gg = wid * NG + g
            sl = pl.ds(gg * L, L)
            ivec = gg * L + iota()
            valid = ivec < n_real
            st = jnp.where(valid, pk_v[sl], 50000 + ivec)
            ln = jnp.where(valid, aux_v[sl], 0)
            pk_v[sl] = (st << 4) | ln
            b = lax.bitcast_convert_type(sm_v[sl], jnp.int32)
            key = jnp.where(b >= 0, b, b ^ np.int32(0x7FFFFFFF))
            key_v[sl] = jnp.where(valid, key, INT_MIN)
            return 0

        lax.fori_loop(0, NG, init_g, 0)
        pltpu.sync_copy(pk_v.at[pl.ds(wid * PT, PT)],
                        sh_pk.at[pl.ds(wid * PT, PT)])
        pltpu.sync_copy(key_v.at[pl.ds(wid * PT, PT)],
                        sh_key.at[pl.ds(wid * PT, PT)])
        plsc.subcore_barrier()
        pltpu.sync_copy(sh_pk, pk_v)
        pltpu.sync_copy(sh_key, key_v)
        plsc.subcore_barrier()

        def srch(g, _):
            base = wid * PT + g * L
            ivec = base + iota()
            sl = pl.ds(base, L)
            pk_i = pk_v[sl]
            s_i = jnp.right_shift(pk_i, 4)
            e_i = s_i + (pk_i & 15)
            t_lo = s_i - 9

            def bstep(p, pos, leq, tgt):
                step = jnp.left_shift(np.int32(1), 14 - p)
                cand = pos + step
                jg = jnp.clip(cand - 1, 0, n_pad - 1)
                s_c = jnp.right_shift(plsc.load_gather(pk_v, [jg]), 4)
                less = (s_c <= tgt) if leq else (s_c < tgt)
                ok = (cand <= n_pad) & less
                return jnp.where(ok, cand, pos)

            def lh_loop(p, st):
                plo, phi = st
                return (bstep(p, plo, False, t_lo), bstep(p, phi, True, e_i))

            z = jnp.zeros((L,), jnp.int32)
            lo, hi = lax.fori_loop(0, 15, lh_loop, (z, z), unroll=3)
            dl = ivec - lo
            dr = hi - ivec - 1
            wlim_v[sl] = (dl << 16) | dr
            lo_w = jnp.clip(jnp.right_shift(base - jnp.max(dl), 5), 0, NW - 1)
            hi_w = jnp.clip(jnp.right_shift(base + 15 + jnp.max(dr), 5),
                            0, NW - 1)
            gg = wid * NG + g
            wb = (gg >> 4) << 4
            lane = gg - wb
            blk = grng_v[pl.ds(wb, L)]
            grng_v[pl.ds(wb, L)] = jnp.where(
                iota() == lane, (lo_w << 16) | hi_w, blk)
            return 0

        lax.fori_loop(0, NG, srch, 0, unroll=2)

        def init_pd(mw, _):
            sl = pl.ds(mw * L, L)
            prev_v[sl] = bits_v[sl]
            return 0

        lax.fori_loop(0, NW // L, init_pd, 0)

        def init_d(gi, _):
            dirty_v[pl.ds(wid * NG + gi * L, L)] = jnp.full((L,), 1, jnp.int32)
            return 0

        lax.fori_loop(0, NG // L, init_d, 0)

        def kp_bits(jc):
            w = plsc.load_gather(bits_v, [jnp.right_shift(jc, 5)])
            return jnp.right_shift(w, jc & 31) & 1

        def sweep(s, prev):
            fb_v[...] = jnp.zeros((L,), jnp.int32)

            @pl.when(prev != 0)
            def _do_sweep():
                def group(g, _):
                    gg = wid * NG + g
                    dirt = rd_word(dirty_v, gg)

                    @pl.when(dirt != 0)
                    def _scan():
                        base = wid * PT + g * L
                        ivec = base + iota()
                        sl = pl.ds(base, L)
                        pk_i = pk_v[sl]
                        s_i = jnp.right_shift(pk_i, 4)
                        e_i = s_i + (pk_i & 15)
                        key_i = key_v[sl]
                        wl = wlim_v[sl]
                        dl = jnp.right_shift(wl, 16)
                        dr = wl & 65535
                        val_i = ivec < n_real

                        def win_body(is_left, d, thr):
                            j = ivec - d if is_left else ivec + d
                            jc = jnp.clip(j, 0, n_pad - 1)
                            pk_j = plsc.load_gather(pk_v, [jc])
                            s_j = jnp.right_shift(pk_j, 4)
                            e_j = s_j + (pk_j & 15)
                            key_j = plsc.load_gather(key_v, [jc])
                            kp = kp_bits(jc)
                            if is_left:
                                inw = d <= dl
                                hit = inw & (s_j < s_i) & (s_i <= e_j) & \
                                    (e_j < e_i) & (key_j >= key_i) & (kp == 1)
                            else:
                                inw = d <= dr
                                hit = inw & (s_j > s_i) & (e_j > e_i) & \
                                    (key_j > key_i) & (kp == 1)
                            return thr | jnp.where(hit, 1, 0)

                        z16 = jnp.zeros((L,), jnp.int32)
                        thr = plsc.parallel_loop(
                            np.int32(1), jnp.max(dl) + 1, unroll=2,
                            carry=z16)(functools.partial(win_body, True))
                        thr = plsc.parallel_loop(
                            np.int32(1), jnp.max(dr) + 1, unroll=2,
                            carry=thr)(functools.partial(win_body, False))
                        new_keep = jnp.where((thr == 0) & val_i, 1, 0)
                        hw = jnp.sum(new_keep << iota())
                        wd = jnp.right_shift(gg, 1)
                        sh = (gg & 1) * 16
                        wb = (wd >> 4) << 4
                        lane = wd - wb
                        blk = bits_v[pl.ds(wb, L)]
                        old = jnp.sum(jnp.where(iota() == lane, blk, 0))
                        neww = (old & ~(65535 << sh)) | (hw << sh)
                        bits_v[pl.ds(wb, L)] = jnp.where(
                            iota() == lane, neww, blk)
                        ch = jnp.where(neww != old, 1, 0)
                        fb_v[...] = fb_v[...] | jnp.full((L,), ch, jnp.int32)

                    return 0

                lax.fori_loop(0, NG, group, 0)

            pltpu.sync_copy(bits_v.at[pl.ds(wid * WPT, WPT)],
                            sh_bits.at[pl.ds(wid * WPT, WPT)])
            pltpu.sync_copy(fb_v.at[pl.ds(0, 8)],
                            sh_flags.at[pl.ds(wid * 8, 8)])
            plsc.subcore_barrier()
            pltpu.sync_copy(sh_bits, bits_v)
            pltpu.sync_copy(sh_flags, fl_v)
            plsc.subcore_barrier()

            @pl.when(prev != 0)
            def _mark_dirty():
                def bld(mw, _):
                    msl = pl.ds(mw * L, L)
                    nv = bits_v[msl]
                    mapfl_v[msl] = jnp.where(nv != prev_v[msl], 1, 0)
                    prev_v[msl] = nv
                    return 0

                lax.fori_loop(0, NW // L, bld, 0)

                def mkd(gi, _):
                    gsl = pl.ds(wid * NG + gi * L, L)
                    rng = grng_v[gsl]
                    lo_w = jnp.right_shift(rng, 16)
                    span = (rng & 65535) - lo_w
                    d = jnp.where(span > 7, 1, 0)
                    for t in range(8):
                        f = plsc.load_gather(
                            mapfl_v, [jnp.clip(lo_w + t, 0, NW - 1)])
                        d = d | jnp.where((t <= span) & (f == 1), 1, 0)
                    dirty_v[gsl] = d
                    return 0

                lax.fori_loop(0, NG // L, mkd, 0)

            def orf(t, a):
                return a | fl_v[pl.ds(t * L, L)]

            vacc = lax.fori_loop(0, 8 * NT // L, orf,
                                 jnp.zeros((L,), jnp.int32))
            return jnp.any(vacc != 0).astype(jnp.int32)

        not_conv = lax.fori_loop(0, S_INNER, sweep, np.int32(1))

        pltpu.sync_copy(bits_v.at[pl.ds(wid * WPT, WPT)],
                        bits_out_hbm.at[pl.ds(wid * WPT, WPT)])
        fb_v[...] = jnp.full((L,), not_conv, jnp.int32)

        @pl.when(wid == 0)
        def _wflag():
            pltpu.sync_copy(fb_v, flag_hbm)

        def unpack16(gg):
            wd = jnp.right_shift(gg, 1)
            sh = (gg & 1) * 16
            w = rd_word(bits_v, wd)
            return jnp.right_shift(w, sh + iota()) & 1

        def cnt_g(g, acc):
            kp = unpack16(wid * NG + g)
            return acc + jnp.sum(kp)

        my_cnt = lax.fori_loop(0, NG, cnt_g, np.int32(0))
        fb_v[...] = jnp.full((L,), my_cnt, jnp.int32)
        pltpu.sync_copy(fb_v.at[pl.ds(0, 8)], sh_flags.at[pl.ds(wid * 8, 8)])
        plsc.subcore_barrier()
        pltpu.sync_copy(sh_flags, fl_v)
        plsc.subcore_barrier()

        def sumf(t, a):
            return a + fl_v[pl.ds(t * L, L)]

        vsum = lax.fori_loop(0, 8 * NT // L, sumf, jnp.zeros((L,), jnp.int32))
        total = jnp.right_shift(jnp.sum(vsum), 3)

        fb_v[...] = jnp.where(iota() < 8, INT_MIN, 0)

        @pl.when(total > kth)
        def _select():
            def level(p, carry):
                rem, hi = carry
                shift = 24 - 8 * p

                def zero_h(w, _):
                    hist_v[pl.ds(w * L, L)] = jnp.zeros((L,), jnp.int32)
                    return 0

                lax.fori_loop(0, 256, zero_h, 0)

                def acc_g(g, _):
                    sl = pl.ds(g * L, L)
                    key = key_v[sl]
                    kp = unpack16(g)
                    pref_ok = jnp.where(
                        p == 0,
                        jnp.ones((L,), jnp.bool_),
                        (key >> jnp.minimum(shift + 8, 31)) == hi)
                    cand = jnp.where((kp == 1) & pref_ok, 1, 0)
                    bn = jnp.where(p == 0, (key >> 24) + 128,
                                   (key >> shift) & 255)
                    plsc.addupdate_scatter(hist_v, [bn * L + iota()], cand)
                    return 0

                lax.fori_loop(0, NGGLOB, acc_g, 0)

                def scan_b(br, st):
                    b = 255 - br
                    found, bstar, acc, rem_n = st
                    hb = jnp.sum(hist_v[pl.ds(b * L, L)])
                    acc2 = acc + hb
                    take = (found == 0) & (acc2 >= rem)
                    bstar = jnp.where(take, b, bstar)
                    rem_n = jnp.where(take, rem - acc, rem_n)
                    found = jnp.where(take, 1, found)
                    return (found, bstar, acc2, rem_n)

                _, bstar, _, rem_n = lax.fori_loop(
                    0, 256, scan_b,
                    (np.int32(0), np.int32(0), np.int32(0), rem))
                bval = jnp.where(p == 0, bstar - 128, bstar)
                return (rem_n, (hi << 8) | bval)

            rem, hi = lax.fori_loop(0, 4, level, (kth, np.int32(0)))
            vstar = hi
            fb_v[...] = jnp.where(iota() < 8, vstar, rem)

            def rank_g(g, c):
                sl = pl.ds(g * L, L)
                key = key_v[sl]
                kp = unpack16(g)
                tie = jnp.where((kp == 1) & (key == vstar), 1, 0)
                incl = jnp.cumsum(tie)
                pk_v[sl] = c + incl - tie
                return c + jnp.sum(tie)

            lax.fori_loop(0, NGGLOB, rank_g, np.int32(0))

        fbv = fb_v[...]
        vstar = fbv[0]
        mquota = fbv[8]

        def out_g(g, _):
            gg = wid * NG + g
            base = gg * L
            sl = pl.ds(base, L)
            kp = unpack16(gg)
            key = key_v[sl]
            rank = pk_v[sl]
            fin = (kp == 1) & ((key > vstar) |
                               ((key == vstar) & (rank < mquota)))
            aux_v[sl] = jnp.where(fin, 1, 0)
            sm_v[sl] = jnp.where(fin, sm_v[sl], 0.0)
            return 0

        lax.fori_loop(0, NG, out_g, 0)
        pltpu.sync_copy(sm_v.at[pl.ds(wid * PT, PT)],
                        out_s_hbm.at[pl.ds(wid * PT, PT)])
        pltpu.sync_copy(aux_v.at[pl.ds(wid * PT, PT)],
                        out_m_hbm.at[pl.ds(wid * PT, PT)])

    return prune


def kernel(g_i, W1, b1, W2, b2, W3, b3, span_start, span_len, T):
    n = g_i.shape[0]
    s_m = _ffnn_scores(g_i, W1, b1, W2, b2, W3) + b3[0]

    n_pad = ((n + 4095) // 4096) * 4096
    st32 = span_start.astype(jnp.int32)
    ln32 = span_len.astype(jnp.int32)
    k = (0.4 * jnp.asarray(T).astype(jnp.float32)).astype(jnp.int32)
    k_arr = jnp.full((16,), k, jnp.int32)

    nw = n_pad // 32
    widx = jnp.arange(nw, dtype=jnp.int32)
    w_last = n // 32
    rem_bits = n % 32
    last_val = (1 << rem_bits) - 1 if rem_bits else 0
    bits0 = jnp.where(widx < w_last, np.int32(-1), np.int32(0))
    bits0 = jnp.where(widx == w_last, np.int32(last_val), bits0)

    prune = _make_sc_prune(n_pad, n)

    def cond(carry):
        return carry[3] != 0

    def body(carry):
        bits, _, _, _ = carry
        out_s, out_m, bits2, flag = prune(s_m, st32, ln32, k_arr, bits)
        return (bits2, out_s, out_m, flag[0])

    _, out_s, out_m, _ = lax.while_loop(
        cond, body,
        (bits0, jnp.zeros((n_pad,), jnp.float32),
         jnp.zeros((n_pad,), jnp.int32), np.int32(1)))
    pruned = out_s[:n]
    mask = out_m[:n].astype(bool)
    return pruned, mask


# speedup vs baseline: 1.2173x; 1.1000x over previous
import functools

import jax
import jax.numpy as jnp
import numpy as np
from jax import lax
from jax.experimental import pallas as pl
from jax.experimental.pallas import tpu as pltpu
from jax.experimental.pallas import tpu_sc as plsc

L = 16
NT = 16
S_INNER = 9
INT_MIN = np.int32(-2147483648)


def _ffnn_body(x_ref, w1_ref, b1_ref, w2_ref, b2_ref, w3_ref, o_ref):
    x = x_ref[...]
    h = jnp.dot(x, w1_ref[...], preferred_element_type=jnp.float32)
    h = jnp.maximum(h + b1_ref[0:1, :], 0.0)
    h = jnp.dot(h, w2_ref[...], preferred_element_type=jnp.float32)
    h = jnp.maximum(h + b2_ref[0:1, :], 0.0)
    o_ref[...] = jnp.dot(h, w3_ref[...], preferred_element_type=jnp.float32)


def _ffnn_scores(g_i, W1, b1, W2, b2, W3):
    n, d_in = g_i.shape
    hid = W1.shape[1]
    HP = 256
    W1p = jnp.zeros((d_in, HP), jnp.float32).at[:, :hid].set(W1)
    b1p = jnp.zeros((8, HP), jnp.float32).at[0, :hid].set(b1)
    W2p = jnp.zeros((HP, HP), jnp.float32).at[:hid, :hid].set(W2)
    b2p = jnp.zeros((8, HP), jnp.float32).at[0, :hid].set(b2)
    W3p = jnp.zeros((HP, 8), jnp.float32).at[:hid, 0].set(W3[:, 0])
    BM = 2000
    assert n % BM == 0
    out = pl.pallas_call(
        _ffnn_body,
        grid=(n // BM,),
        in_specs=[
            pl.BlockSpec((BM, d_in), lambda i: (i, 0)),
            pl.BlockSpec((d_in, HP), lambda i: (0, 0)),
            pl.BlockSpec((8, HP), lambda i: (0, 0)),
            pl.BlockSpec((HP, HP), lambda i: (0, 0)),
            pl.BlockSpec((8, HP), lambda i: (0, 0)),
            pl.BlockSpec((HP, 8), lambda i: (0, 0)),
        ],
        out_specs=pl.BlockSpec((BM, 8), lambda i: (i, 0)),
        out_shape=jax.ShapeDtypeStruct((n, 8), jnp.float32),
    )(g_i, W1p, b1p, W2p, b2p, W3p)
    return out[:, 0]


def _make_sc_prune(n_pad, n_real):
    assert n_pad % (NT * L) == 0
    PT = n_pad // NT
    NG = PT // L
    NW = n_pad // 32
    WPT = NW // NT
    assert WPT % 8 == 0
    NGGLOB = n_pad // L
    iota = lambda: lax.iota(jnp.int32, L)

    mesh = plsc.VectorSubcoreMesh(
        core_axis_name="c", subcore_axis_name="s",
        num_cores=1, num_subcores=NT)

    @functools.partial(
        pl.kernel,
        out_type=[
            jax.ShapeDtypeStruct((n_pad,), jnp.float32),
            jax.ShapeDtypeStruct((n_pad,), jnp.int32),
            jax.ShapeDtypeStruct((NW,), jnp.int32),
            jax.ShapeDtypeStruct((16,), jnp.int32),
        ],
        mesh=mesh,
        compiler_params=pltpu.CompilerParams(needs_layout_passes=False),
        scratch_types=[
            pltpu.VMEM((n_pad,), jnp.int32),
            pltpu.VMEM((n_pad,), jnp.int32),
            pltpu.VMEM((n_pad,), jnp.int32),
            pltpu.VMEM((n_pad,), jnp.float32),
            pltpu.VMEM((n_pad,), jnp.int32),
            pltpu.VMEM((NW,), jnp.int32),
            pltpu.VMEM((NW,), jnp.int32),
            pltpu.VMEM((NW,), jnp.int32),
            pltpu.VMEM((NGGLOB,), jnp.int32),
            pltpu.VMEM((NGGLOB,), jnp.int32),
            pltpu.VMEM((16,), jnp.int32),
            pltpu.VMEM((16,), jnp.int32),
            pltpu.VMEM((8 * NT,), jnp.int32),
            pltpu.VMEM((256 * L,), jnp.int32),
            pltpu.VMEM_SHARED((NW,), jnp.int32),
            pltpu.VMEM_SHARED((8 * NT,), jnp.int32),
            pltpu.VMEM_SHARED((n_pad,), jnp.int32),
            pltpu.VMEM_SHARED((n_pad,), jnp.int32),
        ],
    )
    def prune(sm_hbm, st_hbm, ln_hbm, k_hbm, bits_hbm,
              out_s_hbm, out_m_hbm, bits_out_hbm, flag_hbm,
              pk_v, aux_v, key_v, sm_v, wlim_v, bits_v, prev_v, mapfl_v,
              grng_v, dirty_v, kv_v, fb_v, fl_v, hist_v, sh_bits, sh_flags,
              sh_pk, sh_key):
        wid = lax.axis_index("s") + lax.axis_index("c") * NT

        pltpu.sync_copy(sm_hbm, sm_v.at[pl.ds(0, n_real)])
        pltpu.sync_copy(st_hbm, pk_v.at[pl.ds(0, n_real)])
        pltpu.sync_copy(ln_hbm, aux_v.at[pl.ds(0, n_real)])
        pltpu.sync_copy(k_hbm, kv_v)
        pltpu.sync_copy(bits_hbm, bits_v)
        kth = kv_v[...][0]

        def rd_wvec(ref, wd):
            idx = wd + jnp.zeros((L,), jnp.int32)
            return plsc.load_gather(ref, [idx])

        def rd_word(ref, wd):
            return rd_wvec(ref, wd)[0]

        def init_g(g, _):
            gg = wid * NG + g
            sl = pl.ds(gg * L, L)
            ivec = gg * L + iota()
            valid = ivec < n_real
            st = jnp.where(valid, pk_v[sl], 50000 + ivec)
            ln = jnp.where(valid, aux_v[sl], 0)
            pk_v[sl] = (st << 4) | ln
            b = lax.bitcast_convert_type(sm_v[sl], jnp.int32)
            key = jnp.where(b >= 0, b, b ^ np.int32(0x7FFFFFFF))
            key_v[sl] = jnp.where(valid, key, INT_MIN)
            return 0

        lax.fori_loop(0, NG, init_g, 0)
        pltpu.sync_copy(pk_v.at[pl.ds(wid * PT, PT)],
                        sh_pk.at[pl.ds(wid * PT, PT)])
        pltpu.sync_copy(key_v.at[pl.ds(wid * PT, PT)],
                        sh_key.at[pl.ds(wid * PT, PT)])
        plsc.subcore_barrier()
        pltpu.sync_copy(sh_pk, pk_v)
        pltpu.sync_copy(sh_key, key_v)
        plsc.subcore_barrier()

        def srch(g, _):
            base = wid * PT + g * L
            ivec = base + iota()
            sl = pl.ds(base, L)
            pk_i = pk_v[sl]
            s_i = jnp.right_shift(pk_i, 4)
            e_i = s_i + (pk_i & 15)
            t_lo = s_i - 9

            def bstep(p, pos, leq, tgt):
                step = jnp.left_shift(np.int32(1), 14 - p)
                cand = pos + step
                jg = jnp.clip(cand - 1, 0, n_pad - 1)
                s_c = jnp.right_shift(plsc.load_gather(pk_v, [jg]), 4)
                less = (s_c <= tgt) if leq else (s_c < tgt)
                ok = (cand <= n_pad) & less
                return jnp.where(ok, cand, pos)

            def lh_loop(p, st):
                plo, phi = st
                return (bstep(p, plo, False, t_lo), bstep(p, phi, True, e_i))

            z = jnp.zeros((L,), jnp.int32)
            lo, hi = lax.fori_loop(0, 15, lh_loop, (z, z), unroll=3)
            dl = ivec - lo
            dr = hi - ivec - 1
            wlim_v[sl] = (dl << 16) | dr
            lo_w = jnp.clip(jnp.right_shift(base - jnp.max(dl), 5), 0, NW - 1)
            hi_w = jnp.clip(jnp.right_shift(base + 15 + jnp.max(dr), 5),
                            0, NW - 1)
            gg = wid * NG + g
            wb = (gg >> 4) << 4
            lane = gg - wb
            blk = grng_v[pl.ds(wb, L)]
            grng_v[pl.ds(wb, L)] = jnp.where(
                iota() == lane, (lo_w << 16) | hi_w, blk)
            return 0

        lax.fori_loop(0, NG, srch, 0, unroll=2)

        def init_pd(mw, _):
            sl = pl.ds(mw * L, L)
            prev_v[sl] = bits_v[sl]
            return 0

        lax.fori_loop(0, NW // L, init_pd, 0)

        def init_d(gi, _):
            dirty_v[pl.ds(wid * NG + gi * L, L)] = jnp.full((L,), 1, jnp.int32)
            return 0

        lax.fori_loop(0, NG // L, init_d, 0)

        def kp_bits(jc):
            w = plsc.load_gather(bits_v, [jnp.right_shift(jc, 5)])
            return jnp.right_shift(w, jc & 31) & 1

        def sweep(s, prev):
            fb_v[...] = jnp.zeros((L,), jnp.int32)

            @pl.when(prev != 0)
            def _do_sweep():
                def group(g, _):
                    gg = wid * NG + g
                    dirt = rd_word(dirty_v, gg)

                    @pl.when(dirt != 0)
                    def _scan():
                        base = wid * PT + g * L
                        ivec = base + iota()
                        sl = pl.ds(base, L)
                        pk_i = pk_v[sl]
                        s_i = jnp.right_shift(pk_i, 4)
                        e_i = s_i + (pk_i & 15)
                        key_i = key_v[sl]
                        wl = wlim_v[sl]
                        dl = jnp.right_shift(wl, 16)
                        dr = wl & 65535
                        val_i = ivec < n_real

                        def win_body(d, thr):
                            jl = jnp.clip(ivec - d, 0, n_pad - 1)
                            jr = jnp.clip(ivec + d, 0, n_pad - 1)
                            pk_l = plsc.load_gather(pk_v, [jl])
                            pk_r = plsc.load_gather(pk_v, [jr])
                            key_l = plsc.load_gather(key_v, [jl])
                            key_r = plsc.load_gather(key_v, [jr])
                            kp_l = kp_bits(jl)
                            kp_r = kp_bits(jr)
                            s_l = jnp.right_shift(pk_l, 4)
                            e_l = s_l + (pk_l & 15)
                            s_r = jnp.right_shift(pk_r, 4)
                            e_r = s_r + (pk_r & 15)
                            hit = (d <= dl) & (s_l < s_i) & (s_i <= e_l) & \
                                (e_l < e_i) & (key_l >= key_i) & (kp_l == 1)
                            hit = hit | ((d <= dr) & (s_r > s_i) &
                                         (e_r > e_i) & (key_r > key_i) &
                                         (kp_r == 1))
                            return thr | jnp.where(hit, 1, 0)

                        z16 = jnp.zeros((L,), jnp.int32)
                        dmax = jnp.maximum(jnp.max(dl), jnp.max(dr))
                        thr = plsc.parallel_loop(
                            np.int32(1), dmax + 1, unroll=2,
                            carry=z16)(win_body)
                        new_keep = jnp.where((thr == 0) & val_i, 1, 0)
                        hw = jnp.sum(new_keep << iota())
                        wd = jnp.right_shift(gg, 1)
                        sh = (gg & 1) * 16
                        wb = (wd >> 4) << 4
                        lane = wd - wb
                        blk = bits_v[pl.ds(wb, L)]
                        old = rd_wvec(bits_v, wd)[0]
                        neww = (old & ~(65535 << sh)) | (hw << sh)
                        bits_v[pl.ds(wb, L)] = jnp.where(
                            iota() == lane, neww, blk)
                        ch = jnp.where(neww != old, 1, 0)
                        fb_v[...] = fb_v[...] | jnp.full((L,), ch, jnp.int32)

                    return 0

                lax.fori_loop(0, NG, group, 0)

            pltpu.sync_copy(bits_v.at[pl.ds(wid * WPT, WPT)],
                            sh_bits.at[pl.ds(wid * WPT, WPT)])
            pltpu.sync_copy(fb_v.at[pl.ds(0, 8)],
                            sh_flags.at[pl.ds(wid * 8, 8)])
            plsc.subcore_barrier()
            pltpu.sync_copy(sh_bits, bits_v)
            pltpu.sync_copy(sh_flags, fl_v)
            plsc.subcore_barrier()

            @pl.when(prev != 0)
            def _mark_dirty():
                def bld(mw, _):
                    msl = pl.ds(mw * L, L)
                    nv = bits_v[msl]
                    mapfl_v[msl] = jnp.where(nv != prev_v[msl], 1, 0)
                    prev_v[msl] = nv
                    return 0

                lax.fori_loop(0, NW // L, bld, 0)

                def mkd(gi, _):
                    gsl = pl.ds(wid * NG + gi * L, L)
                    rng = grng_v[gsl]
                    lo_w = jnp.right_shift(rng, 16)
                    span = (rng & 65535) - lo_w
                    d = jnp.where(span > 7, 1, 0)
                    for t in range(8):
                        f = plsc.load_gather(
                            mapfl_v, [jnp.clip(lo_w + t, 0, NW - 1)])
                        d = d | jnp.where((t <= span) & (f == 1), 1, 0)
                    dirty_v[gsl] = d
                    return 0

                lax.fori_loop(0, NG // L, mkd, 0)

            def orf(t, a):
                return a | fl_v[pl.ds(t * L, L)]

            vacc = lax.fori_loop(0, 8 * NT // L, orf,
                                 jnp.zeros((L,), jnp.int32))
            return jnp.any(vacc != 0).astype(jnp.int32)

        not_conv = lax.fori_loop(0, S_INNER, sweep, np.int32(1))

        pltpu.sync_copy(bits_v.at[pl.ds(wid * WPT, WPT)],
                        bits_out_hbm.at[pl.ds(wid * WPT, WPT)])
        fb_v[...] = jnp.full((L,), not_conv, jnp.int32)

        @pl.when(wid == 0)
        def _wflag():
            pltpu.sync_copy(fb_v, flag_hbm)

        def unpack16(gg):
            wd = jnp.right_shift(gg, 1)
            sh = (gg & 1) * 16
            wv = rd_wvec(bits_v, wd)
            return jnp.right_shift(wv, sh + iota()) & 1

        def cnt_g(g, acc):
            kp = unpack16(wid * NG + g)
            return acc + jnp.sum(kp)

        my_cnt = lax.fori_loop(0, NG, cnt_g, np.int32(0))
        fb_v[...] = jnp.full((L,), my_cnt, jnp.int32)
        pltpu.sync_copy(fb_v.at[pl.ds(0, 8)], sh_flags.at[pl.ds(wid * 8, 8)])
        plsc.subcore_barrier()
        pltpu.sync_copy(sh_flags, fl_v)
        plsc.subcore_barrier()

        def sumf(t, a):
            return a + fl_v[pl.ds(t * L, L)]

        vsum = lax.fori_loop(0, 8 * NT // L, sumf, jnp.zeros((L,), jnp.int32))
        total = jnp.right_shift(jnp.sum(vsum), 3)

        fb_v[...] = jnp.where(iota() < 8, INT_MIN, 0)

        @pl.when(total > kth)
        def _select():
            def level(p, carry):
                rem, hi = carry
                shift = 24 - 8 * p

                def zero_h(w, _):
                    hist_v[pl.ds(w * L, L)] = jnp.zeros((L,), jnp.int32)
                    return 0

                lax.fori_loop(0, 256, zero_h, 0)

                def acc_g(g, _):
                    sl = pl.ds(g * L, L)
                    key = key_v[sl]
                    kp = unpack16(g)
                    pref_ok = jnp.where(
                        p == 0,
                        jnp.ones((L,), jnp.bool_),
                        (key >> jnp.minimum(shift + 8, 31)) == hi)
                    cand = jnp.where((kp == 1) & pref_ok, 1, 0)
                    bn = jnp.where(p == 0, (key >> 24) + 128,
                                   (key >> shift) & 255)
                    plsc.addupdate_scatter(hist_v, [bn * L + iota()], cand)
                    return 0

                lax.fori_loop(0, NGGLOB, acc_g, 0)

                def scan_b(br, st):
                    b = 255 - br
                    found, bstar, acc, rem_n = st
                    hb = jnp.sum(hist_v[pl.ds(b * L, L)])
                    acc2 = acc + hb
                    take = (found == 0) & (acc2 >= rem)
                    bstar = jnp.where(take, b, bstar)
                    rem_n = jnp.where(take, rem - acc, rem_n)
                    found = jnp.where(take, 1, found)
                    return (found, bstar, acc2, rem_n)

                _, bstar, _, rem_n = lax.fori_loop(
                    0, 256, scan_b,
                    (np.int32(0), np.int32(0), np.int32(0), rem))
                bval = jnp.where(p == 0, bstar - 128, bstar)
                return (rem_n, (hi << 8) | bval)

            rem, hi = lax.fori_loop(0, 4, level, (kth, np.int32(0)))
            vstar = hi
            fb_v[...] = jnp.where(iota() < 8, vstar, rem)

            def rank_g(g, c):
                sl = pl.ds(g * L, L)
                key = key_v[sl]
                kp = unpack16(g)
                tie = jnp.where((kp == 1) & (key == vstar), 1, 0)
                incl = jnp.cumsum(tie)
                pk_v[sl] = c + incl - tie
                return c + jnp.sum(tie)

            lax.fori_loop(0, NGGLOB, rank_g, np.int32(0))

        fbv = fb_v[...]
        vstar = fbv[0]
        mquota = fbv[8]

        def out_g(g, _):
            gg = wid * NG + g
            base = gg * L
            sl = pl.ds(base, L)
            kp = unpack16(gg)
            key = key_v[sl]
            rank = pk_v[sl]
            fin = (kp == 1) & ((key > vstar) |
                               ((key == vstar) & (rank < mquota)))
            aux_v[sl] = jnp.where(fin, 1, 0)
            sm_v[sl] = jnp.where(fin, sm_v[sl], 0.0)
            return 0

        lax.fori_loop(0, NG, out_g, 0)
        pltpu.sync_copy(sm_v.at[pl.ds(wid * PT, PT)],
                        out_s_hbm.at[pl.ds(wid * PT, PT)])
        pltpu.sync_copy(aux_v.at[pl.ds(wid * PT, PT)],
                        out_m_hbm.at[pl.ds(wid * PT, PT)])

    return prune


def kernel(g_i, W1, b1, W2, b2, W3, b3, span_start, span_len, T):
    n = g_i.shape[0]
    s_m = _ffnn_scores(g_i, W1, b1, W2, b2, W3) + b3[0]

    n_pad = ((n + 4095) // 4096) * 4096
    st32 = span_start.astype(jnp.int32)
    ln32 = span_len.astype(jnp.int32)
    k = (0.4 * jnp.asarray(T).astype(jnp.float32)).astype(jnp.int32)
    k_arr = jnp.full((16,), k, jnp.int32)

    nw = n_pad // 32
    widx = jnp.arange(nw, dtype=jnp.int32)
    w_last = n // 32
    rem_bits = n % 32
    last_val = (1 << rem_bits) - 1 if rem_bits else 0
    bits0 = jnp.where(widx < w_last, np.int32(-1), np.int32(0))
    bits0 = jnp.where(widx == w_last, np.int32(last_val), bits0)

    prune = _make_sc_prune(n_pad, n)

    def cond(carry):
        return carry[3] != 0

    def body(carry):
        bits, _, _, _ = carry
        out_s, out_m, bits2, flag = prune(s_m, st32, ln32, k_arr, bits)
        return (bits2, out_s, out_m, flag[0])

    _, out_s, out_m, _ = lax.while_loop(
        cond, body,
        (bits0, jnp.zeros((n_pad,), jnp.float32),
         jnp.zeros((n_pad,), jnp.int32), np.int32(1)))
    pruned = out_s[:n]
    mask = out_m[:n].astype(bool)
    return pruned, mask


# speedup vs baseline: 1.2343x; 1.0139x over previous
import functools

import jax
import jax.numpy as jnp
import numpy as np
from jax import lax
from jax.experimental import pallas as pl
from jax.experimental.pallas import tpu as pltpu
from jax.experimental.pallas import tpu_sc as plsc

L = 16
NT = 16
S_INNER = 9
INT_MIN = np.int32(-2147483648)


def _ffnn_body(x_ref, w1_ref, b1_ref, w2_ref, b2_ref, w3_ref, o_ref):
    x = x_ref[...]
    h = jnp.dot(x, w1_ref[...], preferred_element_type=jnp.float32)
    h = jnp.maximum(h + b1_ref[0:1, :], 0.0)
    h = jnp.dot(h, w2_ref[...], preferred_element_type=jnp.float32)
    h = jnp.maximum(h + b2_ref[0:1, :], 0.0)
    o_ref[...] = jnp.dot(h, w3_ref[...], preferred_element_type=jnp.float32)


def _ffnn_scores(g_i, W1, b1, W2, b2, W3):
    n, d_in = g_i.shape
    hid = W1.shape[1]
    HP = 256
    W1p = jnp.zeros((d_in, HP), jnp.float32).at[:, :hid].set(W1)
    b1p = jnp.zeros((8, HP), jnp.float32).at[0, :hid].set(b1)
    W2p = jnp.zeros((HP, HP), jnp.float32).at[:hid, :hid].set(W2)
    b2p = jnp.zeros((8, HP), jnp.float32).at[0, :hid].set(b2)
    W3p = jnp.zeros((HP, 8), jnp.float32).at[:hid, 0].set(W3[:, 0])
    BM = 2000
    assert n % BM == 0
    out = pl.pallas_call(
        _ffnn_body,
        grid=(n // BM,),
        in_specs=[
            pl.BlockSpec((BM, d_in), lambda i: (i, 0)),
            pl.BlockSpec((d_in, HP), lambda i: (0, 0)),
            pl.BlockSpec((8, HP), lambda i: (0, 0)),
            pl.BlockSpec((HP, HP), lambda i: (0, 0)),
            pl.BlockSpec((8, HP), lambda i: (0, 0)),
            pl.BlockSpec((HP, 8), lambda i: (0, 0)),
        ],
        out_specs=pl.BlockSpec((BM, 8), lambda i: (i, 0)),
        out_shape=jax.ShapeDtypeStruct((n, 8), jnp.float32),
    )(g_i, W1p, b1p, W2p, b2p, W3p)
    return out[:, 0]


def _make_sc_prune(n_pad, n_real):
    assert n_pad % (NT * L) == 0
    PT = n_pad // NT
    NG = PT // L
    NW = n_pad // 32
    WPT = NW // NT
    assert WPT % 8 == 0
    NGGLOB = n_pad // L
    iota = lambda: lax.iota(jnp.int32, L)

    mesh = plsc.VectorSubcoreMesh(
        core_axis_name="c", subcore_axis_name="s",
        num_cores=1, num_subcores=NT)

    @functools.partial(
        pl.kernel,
        out_type=[
            jax.ShapeDtypeStruct((n_pad,), jnp.float32),
            jax.ShapeDtypeStruct((n_pad,), jnp.int32),
            jax.ShapeDtypeStruct((NW,), jnp.int32),
            jax.ShapeDtypeStruct((16,), jnp.int32),
        ],
        mesh=mesh,
        compiler_params=pltpu.CompilerParams(needs_layout_passes=False),
        scratch_types=[
            pltpu.VMEM((n_pad,), jnp.int32),
            pltpu.VMEM((n_pad,), jnp.int32),
            pltpu.VMEM((n_pad,), jnp.int32),
            pltpu.VMEM((n_pad,), jnp.float32),
            pltpu.VMEM((n_pad,), jnp.int32),
            pltpu.VMEM((NW,), jnp.int32),
            pltpu.VMEM((NW,), jnp.int32),
            pltpu.VMEM((NW,), jnp.int32),
            pltpu.VMEM((NGGLOB,), jnp.int32),
            pltpu.VMEM((NGGLOB,), jnp.int32),
            pltpu.VMEM((16,), jnp.int32),
            pltpu.VMEM((16,), jnp.int32),
            pltpu.VMEM((8 * NT,), jnp.int32),
            pltpu.VMEM((256 * L,), jnp.int32),
            pltpu.SemaphoreType.DMA,
            pltpu.VMEM_SHARED((NW,), jnp.int32),
            pltpu.VMEM_SHARED((8 * NT,), jnp.int32),
            pltpu.VMEM_SHARED((n_pad,), jnp.int32),
            pltpu.VMEM_SHARED((n_pad,), jnp.int32),
        ],
    )
    def prune(sm_hbm, st_hbm, ln_hbm, k_hbm, bits_hbm,
              out_s_hbm, out_m_hbm, bits_out_hbm, flag_hbm,
              pk_v, aux_v, key_v, sm_v, wlim_v, bits_v, prev_v, mapfl_v,
              grng_v, dirty_v, kv_v, fb_v, fl_v, hist_v, stg_sem, sh_bits,
              sh_flags, sh_pk, sh_key):
        wid = lax.axis_index("s") + lax.axis_index("c") * NT

        c1 = pltpu.async_copy(sm_hbm, sm_v.at[pl.ds(0, n_real)], stg_sem)
        c2 = pltpu.async_copy(st_hbm, pk_v.at[pl.ds(0, n_real)], stg_sem)
        c3 = pltpu.async_copy(ln_hbm, aux_v.at[pl.ds(0, n_real)], stg_sem)
        c4 = pltpu.async_copy(k_hbm, kv_v, stg_sem)
        c5 = pltpu.async_copy(bits_hbm, bits_v, stg_sem)
        c1.wait()
        c2.wait()
        c3.wait()
        c4.wait()
        c5.wait()
        kth = kv_v[...][0]

        def rd_wvec(ref, wd):
            idx = wd + jnp.zeros((L,), jnp.int32)
            return plsc.load_gather(ref, [idx])

        def rd_word(ref, wd):
            return rd_wvec(ref, wd)[0]

        def init_g(g, _):
            gg = wid * NG + g
            sl = pl.ds(gg * L, L)
            ivec = gg * L + iota()
            valid = ivec < n_real
            st = jnp.where(valid, pk_v[sl], 50000 + ivec)
            ln = jnp.where(valid, aux_v[sl], 0)
            pk_v[sl] = (st << 4) | ln
            b = lax.bitcast_convert_type(sm_v[sl], jnp.int32)
            key = jnp.where(b >= 0, b, b ^ np.int32(0x7FFFFFFF))
            key_v[sl] = jnp.where(valid, key, INT_MIN)
            return 0

        lax.fori_loop(0, NG, init_g, 0)
        pltpu.sync_copy(pk_v.at[pl.ds(wid * PT, PT)],
                        sh_pk.at[pl.ds(wid * PT, PT)])
        pltpu.sync_copy(key_v.at[pl.ds(wid * PT, PT)],
                        sh_key.at[pl.ds(wid * PT, PT)])
        plsc.subcore_barrier()
        pltpu.sync_copy(sh_pk, pk_v)
        pltpu.sync_copy(sh_key, key_v)
        plsc.subcore_barrier()

        def srch(g, _):
            base = wid * PT + g * L
            ivec = base + iota()
            sl = pl.ds(base, L)
            pk_i = pk_v[sl]
            s_i = jnp.right_shift(pk_i, 4)
            e_i = s_i + (pk_i & 15)
            t_lo = s_i - 9

            def bstep(p, pos, leq, tgt):
                step = jnp.left_shift(np.int32(1), 14 - p)
                cand = pos + step
                jg = jnp.clip(cand - 1, 0, n_pad - 1)
                s_c = jnp.right_shift(plsc.load_gather(pk_v, [jg]), 4)
                less = (s_c <= tgt) if leq else (s_c < tgt)
                ok = (cand <= n_pad) & less
                return jnp.where(ok, cand, pos)

            def lh_loop(p, st):
                plo, phi = st
                return (bstep(p, plo, False, t_lo), bstep(p, phi, True, e_i))

            z = jnp.zeros((L,), jnp.int32)
            lo, hi = lax.fori_loop(0, 15, lh_loop, (z, z), unroll=3)
            dl = ivec - lo
            dr = hi - ivec - 1
            wlim_v[sl] = (dl << 16) | dr
            lo_w = jnp.clip(jnp.right_shift(base - jnp.max(dl), 5), 0, NW - 1)
            hi_w = jnp.clip(jnp.right_shift(base + 15 + jnp.max(dr), 5),
                            0, NW - 1)
            gg = wid * NG + g
            wb = (gg >> 4) << 4
            lane = gg - wb
            blk = grng_v[pl.ds(wb, L)]
            grng_v[pl.ds(wb, L)] = jnp.where(
                iota() == lane, (lo_w << 16) | hi_w, blk)
            return 0

        lax.fori_loop(0, NG, srch, 0, unroll=2)

        def init_pd(mw, _):
            sl = pl.ds(mw * L, L)
            prev_v[sl] = bits_v[sl]
            return 0

        lax.fori_loop(0, NW // L, init_pd, 0)

        def init_d(gi, _):
            dirty_v[pl.ds(wid * NG + gi * L, L)] = jnp.full((L,), 1, jnp.int32)
            return 0

        lax.fori_loop(0, NG // L, init_d, 0)

        def kp_bits(jc):
            w = plsc.load_gather(bits_v, [jnp.right_shift(jc, 5)])
            return jnp.right_shift(w, jc & 31) & 1

        def sweep(s, prev):
            fb_v[...] = jnp.zeros((L,), jnp.int32)

            @pl.when(prev != 0)
            def _do_sweep():
                def group(g, _):
                    gg = wid * NG + g
                    dirt = rd_word(dirty_v, gg)

                    @pl.when(dirt != 0)
                    def _scan():
                        base = wid * PT + g * L
                        ivec = base + iota()
                        sl = pl.ds(base, L)
                        pk_i = pk_v[sl]
                        s_i = jnp.right_shift(pk_i, 4)
                        e_i = s_i + (pk_i & 15)
                        key_i = key_v[sl]
                        wl = wlim_v[sl]
                        dl = jnp.right_shift(wl, 16)
                        dr = wl & 65535
                        val_i = ivec < n_real

                        def win_body(d, thr):
                            jl = jnp.clip(ivec - d, 0, n_pad - 1)
                            jr = jnp.clip(ivec + d, 0, n_pad - 1)
                            pk_l = plsc.load_gather(pk_v, [jl])
                            pk_r = plsc.load_gather(pk_v, [jr])
                            key_l = plsc.load_gather(key_v, [jl])
                            key_r = plsc.load_gather(key_v, [jr])
                            kp_l = kp_bits(jl)
                            kp_r = kp_bits(jr)
                            s_l = jnp.right_shift(pk_l, 4)
                            e_l = s_l + (pk_l & 15)
                            s_r = jnp.right_shift(pk_r, 4)
                            e_r = s_r + (pk_r & 15)
                            hit = (d <= dl) & (s_l < s_i) & (s_i <= e_l) & \
                                (e_l < e_i) & (key_l >= key_i) & (kp_l == 1)
                            hit = hit | ((d <= dr) & (s_r > s_i) &
                                         (e_r > e_i) & (key_r > key_i) &
                                         (kp_r == 1))
                            return thr | jnp.where(hit, 1, 0)

                        z16 = jnp.zeros((L,), jnp.int32)
                        dmax = jnp.maximum(jnp.max(dl), jnp.max(dr))
                        thr = plsc.parallel_loop(
                            np.int32(1), dmax + 1, unroll=4,
                            carry=z16)(win_body)
                        new_keep = jnp.where((thr == 0) & val_i, 1, 0)
                        hw = jnp.sum(new_keep << iota())
                        wd = jnp.right_shift(gg, 1)
                        sh = (gg & 1) * 16
                        wb = (wd >> 4) << 4
                        lane = wd - wb
                        blk = bits_v[pl.ds(wb, L)]
                        old = rd_wvec(bits_v, wd)[0]
                        neww = (old & ~(65535 << sh)) | (hw << sh)
                        bits_v[pl.ds(wb, L)] = jnp.where(
                            iota() == lane, neww, blk)
                        ch = jnp.where(neww != old, 1, 0)
                        fb_v[...] = fb_v[...] | jnp.full((L,), ch, jnp.int32)

                    return 0

                lax.fori_loop(0, NG, group, 0)

            pltpu.sync_copy(bits_v.at[pl.ds(wid * WPT, WPT)],
                            sh_bits.at[pl.ds(wid * WPT, WPT)])
            pltpu.sync_copy(fb_v.at[pl.ds(0, 8)],
                            sh_flags.at[pl.ds(wid * 8, 8)])
            plsc.subcore_barrier()
            pltpu.sync_copy(sh_bits, bits_v)
            pltpu.sync_copy(sh_flags, fl_v)
            plsc.subcore_barrier()

            @pl.when(prev != 0)
            def _mark_dirty():
                def bld(mw, _):
                    msl = pl.ds(mw * L, L)
                    nv = bits_v[msl]
                    mapfl_v[msl] = jnp.where(nv != prev_v[msl], 1, 0)
                    prev_v[msl] = nv
                    return 0

                lax.fori_loop(0, NW // L, bld, 0)

                def mkd(gi, _):
                    gsl = pl.ds(wid * NG + gi * L, L)
                    rng = grng_v[gsl]
                    lo_w = jnp.right_shift(rng, 16)
                    span = (rng & 65535) - lo_w
                    d = jnp.where(span > 7, 1, 0)
                    for t in range(8):
                        f = plsc.load_gather(
                            mapfl_v, [jnp.clip(lo_w + t, 0, NW - 1)])
                        d = d | jnp.where((t <= span) & (f == 1), 1, 0)
                    dirty_v[gsl] = d
                    return 0

                lax.fori_loop(0, NG // L, mkd, 0)

            def orf(t, a):
                return a | fl_v[pl.ds(t * L, L)]

            vacc = lax.fori_loop(0, 8 * NT // L, orf,
                                 jnp.zeros((L,), jnp.int32))
            return jnp.any(vacc != 0).astype(jnp.int32)

        not_conv = lax.fori_loop(0, S_INNER, sweep, np.int32(1))

        pltpu.sync_copy(bits_v.at[pl.ds(wid * WPT, WPT)],
                        bits_out_hbm.at[pl.ds(wid * WPT, WPT)])
        fb_v[...] = jnp.full((L,), not_conv, jnp.int32)

        @pl.when(wid == 0)
        def _wflag():
            pltpu.sync_copy(fb_v, flag_hbm)

        def unpack16(gg):
            wd = jnp.right_shift(gg, 1)
            sh = (gg & 1) * 16
            wv = rd_wvec(bits_v, wd)
            return jnp.right_shift(wv, sh + iota()) & 1

        def cnt_g(g, acc):
            kp = unpack16(wid * NG + g)
            return acc + jnp.sum(kp)

        my_cnt = lax.fori_loop(0, NG, cnt_g, np.int32(0))
        fb_v[...] = jnp.full((L,), my_cnt, jnp.int32)
        pltpu.sync_copy(fb_v.at[pl.ds(0, 8)], sh_flags.at[pl.ds(wid * 8, 8)])
        plsc.subcore_barrier()
        pltpu.sync_copy(sh_flags, fl_v)
        plsc.subcore_barrier()

        def sumf(t, a):
            return a + fl_v[pl.ds(t * L, L)]

        vsum = lax.fori_loop(0, 8 * NT // L, sumf, jnp.zeros((L,), jnp.int32))
        total = jnp.right_shift(jnp.sum(vsum), 3)

        fb_v[...] = jnp.where(iota() < 8, INT_MIN, 0)

        @pl.when(total > kth)
        def _select():
            def level(p, carry):
                rem, hi = carry
                shift = 24 - 8 * p

                def zero_h(w, _):
                    hist_v[pl.ds(w * L, L)] = jnp.zeros((L,), jnp.int32)
                    return 0

                lax.fori_loop(0, 256, zero_h, 0)

                def acc_g(g, _):
                    sl = pl.ds(g * L, L)
                    key = key_v[sl]
                    kp = unpack16(g)
                    pref_ok = jnp.where(
                        p == 0,
                        jnp.ones((L,), jnp.bool_),
                        (key >> jnp.minimum(shift + 8, 31)) == hi)
                    cand = jnp.where((kp == 1) & pref_ok, 1, 0)
                    bn = jnp.where(p == 0, (key >> 24) + 128,
                                   (key >> shift) & 255)
                    plsc.addupdate_scatter(hist_v, [bn * L + iota()], cand)
                    return 0

                lax.fori_loop(0, NGGLOB, acc_g, 0)

                def scan_b(br, st):
                    b = 255 - br
                    found, bstar, acc, rem_n = st
                    hb = jnp.sum(hist_v[pl.ds(b * L, L)])
                    acc2 = acc + hb
                    take = (found == 0) & (acc2 >= rem)
                    bstar = jnp.where(take, b, bstar)
                    rem_n = jnp.where(take, rem - acc, rem_n)
                    found = jnp.where(take, 1, found)
                    return (found, bstar, acc2, rem_n)

                _, bstar, _, rem_n = lax.fori_loop(
                    0, 256, scan_b,
                    (np.int32(0), np.int32(0), np.int32(0), rem))
                bval = jnp.where(p == 0, bstar - 128, bstar)
                return (rem_n, (hi << 8) | bval)

            rem, hi = lax.fori_loop(0, 4, level, (kth, np.int32(0)))
            vstar = hi
            fb_v[...] = jnp.where(iota() < 8, vstar, rem)

            def rank_g(g, c):
                sl = pl.ds(g * L, L)
                key = key_v[sl]
                kp = unpack16(g)
                tie = jnp.where((kp == 1) & (key == vstar), 1, 0)
                incl = jnp.cumsum(tie)
                pk_v[sl] = c + incl - tie
                return c + jnp.sum(tie)

            lax.fori_loop(0, NGGLOB, rank_g, np.int32(0))

        fbv = fb_v[...]
        vstar = fbv[0]
        mquota = fbv[8]

        def out_g(g, _):
            gg = wid * NG + g
            base = gg * L
            sl = pl.ds(base, L)
            kp = unpack16(gg)
            key = key_v[sl]
            rank = pk_v[sl]
            fin = (kp == 1) & ((key > vstar) |
                               ((key == vstar) & (rank < mquota)))
            aux_v[sl] = jnp.where(fin, 1, 0)
            sm_v[sl] = jnp.where(fin, sm_v[sl], 0.0)
            return 0

        lax.fori_loop(0, NG, out_g, 0)
        pltpu.sync_copy(sm_v.at[pl.ds(wid * PT, PT)],
                        out_s_hbm.at[pl.ds(wid * PT, PT)])
        pltpu.sync_copy(aux_v.at[pl.ds(wid * PT, PT)],
                        out_m_hbm.at[pl.ds(wid * PT, PT)])

    return prune


def kernel(g_i, W1, b1, W2, b2, W3, b3, span_start, span_len, T):
    n = g_i.shape[0]
    s_m = _ffnn_scores(g_i, W1, b1, W2, b2, W3) + b3[0]

    n_pad = ((n + 4095) // 4096) * 4096
    st32 = span_start.astype(jnp.int32)
    ln32 = span_len.astype(jnp.int32)
    k = (0.4 * jnp.asarray(T).astype(jnp.float32)).astype(jnp.int32)
    k_arr = jnp.full((16,), k, jnp.int32)

    nw = n_pad // 32
    widx = jnp.arange(nw, dtype=jnp.int32)
    w_last = n // 32
    rem_bits = n % 32
    last_val = (1 << rem_bits) - 1 if rem_bits else 0
    bits0 = jnp.where(widx < w_last, np.int32(-1), np.int32(0))
    bits0 = jnp.where(widx == w_last, np.int32(last_val), bits0)

    prune = _make_sc_prune(n_pad, n)

    def cond(carry):
        return carry[3] != 0

    def body(carry):
        bits, _, _, _ = carry
        out_s, out_m, bits2, flag = prune(s_m, st32, ln32, k_arr, bits)
        return (bits2, out_s, out_m, flag[0])

    _, out_s, out_m, _ = lax.while_loop(
        cond, body,
        (bits0, jnp.zeros((n_pad,), jnp.float32),
         jnp.zeros((n_pad,), jnp.int32), np.int32(1)))
    pruned = out_s[:n]
    mask = out_m[:n].astype(bool)
    return pruned, mask


# speedup vs baseline: 1.2654x; 1.0252x over previous
import functools

import jax
import jax.numpy as jnp
import numpy as np
from jax import lax
from jax.experimental import pallas as pl
from jax.experimental.pallas import tpu as pltpu
from jax.experimental.pallas import tpu_sc as plsc

L = 16
NT = 16
S_INNER = 9
INT_MIN = np.int32(-2147483648)


def _ffnn_body(x_ref, w1_ref, b1_ref, w2_ref, b2_ref, w3_ref, o_ref):
    x = x_ref[...]
    h = jnp.dot(x, w1_ref[...], preferred_element_type=jnp.float32)
    h = jnp.maximum(h + b1_ref[0:1, :], 0.0)
    h = jnp.dot(h, w2_ref[...], preferred_element_type=jnp.float32)
    h = jnp.maximum(h + b2_ref[0:1, :], 0.0)
    o_ref[...] = jnp.dot(h, w3_ref[...], preferred_element_type=jnp.float32)


def _ffnn_scores(g_i, W1, b1, W2, b2, W3):
    n, d_in = g_i.shape
    hid = W1.shape[1]
    HP = 256
    W1p = jnp.zeros((d_in, HP), jnp.float32).at[:, :hid].set(W1)
    b1p = jnp.zeros((8, HP), jnp.float32).at[0, :hid].set(b1)
    W2p = jnp.zeros((HP, HP), jnp.float32).at[:hid, :hid].set(W2)
    b2p = jnp.zeros((8, HP), jnp.float32).at[0, :hid].set(b2)
    W3p = jnp.zeros((HP, 8), jnp.float32).at[:hid, 0].set(W3[:, 0])
    BM = 2000
    assert n % BM == 0
    out = pl.pallas_call(
        _ffnn_body,
        grid=(n // BM,),
        in_specs=[
            pl.BlockSpec((BM, d_in), lambda i: (i, 0)),
            pl.BlockSpec((d_in, HP), lambda i: (0, 0)),
            pl.BlockSpec((8, HP), lambda i: (0, 0)),
            pl.BlockSpec((HP, HP), lambda i: (0, 0)),
            pl.BlockSpec((8, HP), lambda i: (0, 0)),
            pl.BlockSpec((HP, 8), lambda i: (0, 0)),
        ],
        out_specs=pl.BlockSpec((BM, 8), lambda i: (i, 0)),
        out_shape=jax.ShapeDtypeStruct((n, 8), jnp.float32),
    )(g_i, W1p, b1p, W2p, b2p, W3p)
    return out[:, 0]


def _make_sc_prune(n_pad, n_real):
    assert n_pad % (NT * L) == 0
    PT = n_pad // NT
    NG = PT // L
    NW = n_pad // 32
    WPT = NW // NT
    assert WPT % 8 == 0
    NGGLOB = n_pad // L
    iota = lambda: lax.iota(jnp.int32, L)

    mesh = plsc.VectorSubcoreMesh(
        core_axis_name="c", subcore_axis_name="s",
        num_cores=1, num_subcores=NT)

    @functools.partial(
        pl.kernel,
        out_type=[
            jax.ShapeDtypeStruct((n_pad,), jnp.float32),
            jax.ShapeDtypeStruct((n_pad,), jnp.int32),
            jax.ShapeDtypeStruct((NW,), jnp.int32),
            jax.ShapeDtypeStruct((16,), jnp.int32),
        ],
        mesh=mesh,
        compiler_params=pltpu.CompilerParams(needs_layout_passes=False),
        scratch_types=[
            pltpu.VMEM((n_pad,), jnp.int32),
            pltpu.VMEM((n_pad,), jnp.int32),
            pltpu.VMEM((n_pad,), jnp.int32),
            pltpu.VMEM((n_pad,), jnp.float32),
            pltpu.VMEM((n_pad,), jnp.int32),
            pltpu.VMEM((NW,), jnp.int32),
            pltpu.VMEM((NW,), jnp.int32),
            pltpu.VMEM((NW,), jnp.int32),
            pltpu.VMEM((NGGLOB,), jnp.int32),
            pltpu.VMEM((NGGLOB,), jnp.int32),
            pltpu.VMEM((16,), jnp.int32),
            pltpu.VMEM((16,), jnp.int32),
            pltpu.VMEM((8 * NT,), jnp.int32),
            pltpu.VMEM((256 * L,), jnp.int32),
            pltpu.SemaphoreType.DMA,
            pltpu.VMEM_SHARED((NW,), jnp.int32),
            pltpu.VMEM_SHARED((8 * NT,), jnp.int32),
            pltpu.VMEM_SHARED((n_pad,), jnp.int32),
            pltpu.VMEM_SHARED((n_pad,), jnp.int32),
        ],
    )
    def prune(sm_hbm, st_hbm, ln_hbm, k_hbm, bits_hbm,
              out_s_hbm, out_m_hbm, bits_out_hbm, flag_hbm,
              pk_v, aux_v, key_v, sm_v, wlim_v, bits_v, prev_v, mapfl_v,
              grng_v, dirty_v, kv_v, fb_v, fl_v, hist_v, stg_sem, sh_bits,
              sh_flags, sh_pk, sh_key):
        wid = lax.axis_index("s") + lax.axis_index("c") * NT

        c1 = pltpu.async_copy(sm_hbm, sm_v.at[pl.ds(0, n_real)], stg_sem)
        c2 = pltpu.async_copy(st_hbm, pk_v.at[pl.ds(0, n_real)], stg_sem)
        c3 = pltpu.async_copy(ln_hbm, aux_v.at[pl.ds(0, n_real)], stg_sem)
        c4 = pltpu.async_copy(k_hbm, kv_v, stg_sem)
        c5 = pltpu.async_copy(bits_hbm, bits_v, stg_sem)
        c1.wait()
        c2.wait()
        c3.wait()
        c4.wait()
        c5.wait()
        kth = kv_v[...][0]

        def rd_wvec(ref, wd):
            idx = wd + jnp.zeros((L,), jnp.int32)
            return plsc.load_gather(ref, [idx])

        def rd_word(ref, wd):
            return rd_wvec(ref, wd)[0]

        def init_g(g, _):
            gg = wid * NG + g
            sl = pl.ds(gg * L, L)
            ivec = gg * L + iota()
            valid = ivec < n_real
            st = jnp.where(valid, pk_v[sl], 50000 + ivec)
            ln = jnp.where(valid, aux_v[sl], 0)
            pk_v[sl] = (st << 4) | ln
            b = lax.bitcast_convert_type(sm_v[sl], jnp.int32)
            key = jnp.where(b >= 0, b, b ^ np.int32(0x7FFFFFFF))
            key_v[sl] = jnp.where(valid, key, INT_MIN)
            return 0

        lax.fori_loop(0, NG, init_g, 0)
        pltpu.sync_copy(pk_v.at[pl.ds(wid * PT, PT)],
                        sh_pk.at[pl.ds(wid * PT, PT)])
        pltpu.sync_copy(key_v.at[pl.ds(wid * PT, PT)],
                        sh_key.at[pl.ds(wid * PT, PT)])
        plsc.subcore_barrier()
        pltpu.sync_copy(sh_pk, pk_v)
        pltpu.sync_copy(sh_key, key_v)
        plsc.subcore_barrier()

        def srch(g, _):
            base = wid * PT + g * L
            ivec = base + iota()
            sl = pl.ds(base, L)
            pk_i = pk_v[sl]
            s_i = jnp.right_shift(pk_i, 4)
            e_i = s_i + (pk_i & 15)
            t_lo = s_i - 9

            def bstep(p, pos, leq, tgt):
                step = jnp.left_shift(np.int32(1), 14 - p)
                cand = pos + step
                jg = jnp.clip(cand - 1, 0, n_pad - 1)
                s_c = jnp.right_shift(plsc.load_gather(pk_v, [jg]), 4)
                less = (s_c <= tgt) if leq else (s_c < tgt)
                ok = (cand <= n_pad) & less
                return jnp.where(ok, cand, pos)

            def lh_loop(p, st):
                plo, phi = st
                return (bstep(p, plo, False, t_lo), bstep(p, phi, True, e_i))

            z = jnp.zeros((L,), jnp.int32)
            lo, hi = lax.fori_loop(0, 15, lh_loop, (z, z), unroll=3)
            dl = ivec - lo
            dr = hi - ivec - 1
            wlim_v[sl] = (dl << 16) | dr
            lo_w = jnp.clip(jnp.right_shift(base - jnp.max(dl), 5), 0, NW - 1)
            hi_w = jnp.clip(jnp.right_shift(base + 15 + jnp.max(dr), 5),
                            0, NW - 1)
            gg = wid * NG + g
            wb = (gg >> 4) << 4
            lane = gg - wb
            blk = grng_v[pl.ds(wb, L)]
            grng_v[pl.ds(wb, L)] = jnp.where(
                iota() == lane, (lo_w << 16) | hi_w, blk)
            return 0

        lax.fori_loop(0, NG, srch, 0, unroll=2)

        def init_pd(mw, _):
            sl = pl.ds(mw * L, L)
            prev_v[sl] = bits_v[sl]
            return 0

        lax.fori_loop(0, NW // L, init_pd, 0)

        def init_d(gi, _):
            dirty_v[pl.ds(wid * NG + gi * L, L)] = jnp.full((L,), 1, jnp.int32)
            return 0

        lax.fori_loop(0, NG // L, init_d, 0)

        def kp_bits(jc):
            w = plsc.load_gather(bits_v, [jnp.right_shift(jc, 5)])
            return jnp.right_shift(w, jc & 31) & 1

        def sweep(s, prev):
            fb_v[...] = jnp.zeros((L,), jnp.int32)

            @pl.when(prev != 0)
            def _do_sweep():
                def group(g, _):
                    gg = wid * NG + g
                    dirt = rd_word(dirty_v, gg)

                    @pl.when(dirt != 0)
                    def _scan():
                        base = wid * PT + g * L
                        ivec = base + iota()
                        sl = pl.ds(base, L)
                        pk_i = pk_v[sl]
                        s_i = jnp.right_shift(pk_i, 4)
                        e_i = s_i + (pk_i & 15)
                        key_i = key_v[sl]
                        wl = wlim_v[sl]
                        dl = jnp.right_shift(wl, 16)
                        dr = wl & 65535
                        val_i = ivec < n_real

                        def win_body(d, thr):
                            jl = jnp.clip(ivec - d, 0, n_pad - 1)
                            jr = jnp.clip(ivec + d, 0, n_pad - 1)
                            pk_l = plsc.load_gather(pk_v, [jl])
                            pk_r = plsc.load_gather(pk_v, [jr])
                            key_l = plsc.load_gather(key_v, [jl])
                            key_r = plsc.load_gather(key_v, [jr])
                            kp_l = kp_bits(jl)
                            kp_r = kp_bits(jr)
                            s_l = jnp.right_shift(pk_l, 4)
                            e_l = s_l + (pk_l & 15)
                            s_r = jnp.right_shift(pk_r, 4)
                            e_r = s_r + (pk_r & 15)
                            hit = (d <= dl) & (s_l < s_i) & (s_i <= e_l) & \
                                (e_l < e_i) & (key_l >= key_i) & (kp_l == 1)
                            hit = hit | ((d <= dr) & (s_r > s_i) &
                                         (e_r > e_i) & (key_r > key_i) &
                                         (kp_r == 1))
                            return thr | jnp.where(hit, 1, 0)

                        z16 = jnp.zeros((L,), jnp.int32)
                        dmax = jnp.maximum(jnp.max(dl), jnp.max(dr))
                        thr = plsc.parallel_loop(
                            np.int32(1), dmax + 1, unroll=2,
                            carry=z16)(win_body)
                        new_keep = jnp.where((thr == 0) & val_i, 1, 0)
                        hw = jnp.sum(new_keep << iota())
                        wd = jnp.right_shift(gg, 1)
                        sh = (gg & 1) * 16
                        wb = (wd >> 4) << 4
                        lane = wd - wb
                        blk = bits_v[pl.ds(wb, L)]
                        old = rd_wvec(bits_v, wd)[0]
                        neww = (old & ~(65535 << sh)) | (hw << sh)
                        bits_v[pl.ds(wb, L)] = jnp.where(
                            iota() == lane, neww, blk)
                        ch = jnp.where(neww != old, 1, 0)
                        fb_v[...] = fb_v[...] | jnp.full((L,), ch, jnp.int32)

                    return 0

                lax.fori_loop(0, NG, group, 0)

            pltpu.sync_copy(bits_v.at[pl.ds(wid * WPT, WPT)],
                            sh_bits.at[pl.ds(wid * WPT, WPT)])
            pltpu.sync_copy(fb_v.at[pl.ds(0, 8)],
                            sh_flags.at[pl.ds(wid * 8, 8)])
            plsc.subcore_barrier()
            pltpu.sync_copy(sh_bits, bits_v)
            pltpu.sync_copy(sh_flags, fl_v)
            plsc.subcore_barrier()

            @pl.when(prev != 0)
            def _mark_dirty():
                def bld(mw, _):
                    msl = pl.ds(mw * L, L)
                    nv = bits_v[msl]
                    mapfl_v[msl] = jnp.where(nv != prev_v[msl], 1, 0)
                    prev_v[msl] = nv
                    return 0

                lax.fori_loop(0, NW // L, bld, 0)

                def mkd(gi, _):
                    gsl = pl.ds(wid * NG + gi * L, L)
                    rng = grng_v[gsl]
                    lo_w = jnp.right_shift(rng, 16)
                    span = (rng & 65535) - lo_w
                    d = jnp.where(span > 7, 1, 0)
                    for t in range(8):
                        f = plsc.load_gather(
                            mapfl_v, [jnp.clip(lo_w + t, 0, NW - 1)])
                        d = d | jnp.where((t <= span) & (f == 1), 1, 0)
                    dirty_v[gsl] = d
                    return 0

                lax.fori_loop(0, NG // L, mkd, 0)

            def orf(t, a):
                return a | fl_v[pl.ds(t * L, L)]

            vacc = lax.fori_loop(0, 8 * NT // L, orf,
                                 jnp.zeros((L,), jnp.int32))
            return jnp.any(vacc != 0).astype(jnp.int32)

        not_conv = lax.fori_loop(0, S_INNER, sweep, np.int32(1))

        pltpu.sync_copy(bits_v.at[pl.ds(wid * WPT, WPT)],
                        bits_out_hbm.at[pl.ds(wid * WPT, WPT)])
        fb_v[...] = jnp.full((L,), not_conv, jnp.int32)

        @pl.when(wid == 0)
        def _wflag():
            pltpu.sync_copy(fb_v, flag_hbm)

        def unpack16(gg):
            wd = jnp.right_shift(gg, 1)
            sh = (gg & 1) * 16
            wv = rd_wvec(bits_v, wd)
            return jnp.right_shift(wv, sh + iota()) & 1

        def cnt_g(g, acc):
            kp = unpack16(wid * NG + g)
            return acc + jnp.sum(kp)

        my_cnt = lax.fori_loop(0, NG, cnt_g, np.int32(0))
        fb_v[...] = jnp.full((L,), my_cnt, jnp.int32)
        pltpu.sync_copy(fb_v.at[pl.ds(0, 8)], sh_flags.at[pl.ds(wid * 8, 8)])
        plsc.subcore_barrier()
        pltpu.sync_copy(sh_flags, fl_v)
        plsc.subcore_barrier()

        def sumf(t, a):
            return a + fl_v[pl.ds(t * L, L)]

        vsum = lax.fori_loop(0, 8 * NT // L, sumf, jnp.zeros((L,), jnp.int32))
        total = jnp.right_shift(jnp.sum(vsum), 3)

        fb_v[...] = jnp.where(iota() < 8, INT_MIN, 0)

        @pl.when(total > kth)
        def _select():
            def level(p, carry):
                rem, hi = carry
                shift = 24 - 8 * p

                def zero_h(w, _):
                    hist_v[pl.ds(w * L, L)] = jnp.zeros((L,), jnp.int32)
                    return 0

                lax.fori_loop(0, 256, zero_h, 0)

                def acc_g(g, _):
                    sl = pl.ds(g * L, L)
                    key = key_v[sl]
                    kp = unpack16(g)
                    pref_ok = jnp.where(
                        p == 0,
                        jnp.ones((L,), jnp.bool_),
                        (key >> jnp.minimum(shift + 8, 31)) == hi)
                    cand = jnp.where((kp == 1) & pref_ok, 1, 0)
                    bn = jnp.where(p == 0, (key >> 24) + 128,
                                   (key >> shift) & 255)
                    plsc.addupdate_scatter(hist_v, [bn * L + iota()], cand)
                    return 0

                lax.fori_loop(0, NGGLOB, acc_g, 0)

                def scan_b(br, st):
                    b = 255 - br
                    found, bstar, acc, rem_n = st
                    hb = jnp.sum(hist_v[pl.ds(b * L, L)])
                    acc2 = acc + hb
                    take = (found == 0) & (acc2 >= rem)
                    bstar = jnp.where(take, b, bstar)
                    rem_n = jnp.where(take, rem - acc, rem_n)
                    found = jnp.where(take, 1, found)
                    return (found, bstar, acc2, rem_n)

                _, bstar, _, rem_n = lax.fori_loop(
                    0, 256, scan_b,
                    (np.int32(0), np.int32(0), np.int32(0), rem))
                bval = jnp.where(p == 0, bstar - 128, bstar)
                return (rem_n, (hi << 8) | bval)

            rem, hi = lax.fori_loop(0, 4, level, (kth, np.int32(0)))
            vstar = hi
            fb_v[...] = jnp.where(iota() < 8, vstar, rem)

            def rank_g(g, c):
                sl = pl.ds(g * L, L)
                key = key_v[sl]
                kp = unpack16(g)
                tie = jnp.where((kp == 1) & (key == vstar), 1, 0)
                incl = jnp.cumsum(tie)
                pk_v[sl] = c + incl - tie
                return c + jnp.sum(tie)

            lax.fori_loop(0, NGGLOB, rank_g, np.int32(0))

        fbv = fb_v[...]
        vstar = fbv[0]
        mquota = fbv[8]

        def out_g(g, _):
            gg = wid * NG + g
            base = gg * L
            sl = pl.ds(base, L)
            kp = unpack16(gg)
            key = key_v[sl]
            rank = pk_v[sl]
            fin = (kp == 1) & ((key > vstar) |
                               ((key == vstar) & (rank < mquota)))
            aux_v[sl] = jnp.where(fin, 1, 0)
            sm_v[sl] = jnp.where(fin, sm_v[sl], 0.0)
            return 0

        lax.fori_loop(0, NG, out_g, 0)
        pltpu.sync_copy(sm_v.at[pl.ds(wid * PT, PT)],
                        out_s_hbm.at[pl.ds(wid * PT, PT)])
        pltpu.sync_copy(aux_v.at[pl.ds(wid * PT, PT)],
                        out_m_hbm.at[pl.ds(wid * PT, PT)])

    return prune


def kernel(g_i, W1, b1, W2, b2, W3, b3, span_start, span_len, T):
    n = g_i.shape[0]
    s_m = _ffnn_scores(g_i, W1, b1, W2, b2, W3) + b3[0]

    n_pad = ((n + 4095) // 4096) * 4096
    st32 = span_start.astype(jnp.int32)
    ln32 = span_len.astype(jnp.int32)
    k = (0.4 * jnp.asarray(T).astype(jnp.float32)).astype(jnp.int32)
    k_arr = jnp.full((16,), k, jnp.int32)

    nw = n_pad // 32
    widx = jnp.arange(nw, dtype=jnp.int32)
    w_last = n // 32
    rem_bits = n % 32
    last_val = (1 << rem_bits) - 1 if rem_bits else 0
    bits0 = jnp.where(widx < w_last, np.int32(-1), np.int32(0))
    bits0 = jnp.where(widx == w_last, np.int32(last_val), bits0)

    prune = _make_sc_prune(n_pad, n)

    def cond(carry):
        return carry[3] != 0

    def body(carry):
        bits, _, _, _ = carry
        out_s, out_m, bits2, flag = prune(s_m, st32, ln32, k_arr, bits)
        return (bits2, out_s, out_m, flag[0])

    _, out_s, out_m, _ = lax.while_loop(
        cond, body,
        (bits0, jnp.zeros((n_pad,), jnp.float32),
         jnp.zeros((n_pad,), jnp.int32), np.int32(1)))
    pruned = out_s[:n]
    mask = out_m[:n].astype(bool)
    return pruned, mask


# speedup vs baseline: 1.2953x; 1.0236x over previous
import functools

import jax
import jax.numpy as jnp
import numpy as np
from jax import lax
from jax.experimental import pallas as pl
from jax.experimental.pallas import tpu as pltpu
from jax.experimental.pallas import tpu_sc as plsc

L = 16
NT = 16
S_INNER = 9
INT_MIN = np.int32(-2147483648)


def _ffnn_body(x_ref, w1_ref, b1_ref, w2_ref, b2_ref, w3_ref, o_ref):
    x = x_ref[...]
    h = jnp.dot(x, w1_ref[...], preferred_element_type=jnp.float32)
    h = jnp.maximum(h + b1_ref[0:1, :], 0.0)
    h = jnp.dot(h, w2_ref[...], preferred_element_type=jnp.float32)
    h = jnp.maximum(h + b2_ref[0:1, :], 0.0)
    o_ref[...] = jnp.dot(h, w3_ref[...], preferred_element_type=jnp.float32)


def _ffnn_scores(g_i, W1, b1, W2, b2, W3):
    n, d_in = g_i.shape
    hid = W1.shape[1]
    HP = 256
    W1p = jnp.zeros((d_in, HP), jnp.float32).at[:, :hid].set(W1)
    b1p = jnp.zeros((8, HP), jnp.float32).at[0, :hid].set(b1)
    W2p = jnp.zeros((HP, HP), jnp.float32).at[:hid, :hid].set(W2)
    b2p = jnp.zeros((8, HP), jnp.float32).at[0, :hid].set(b2)
    W3p = jnp.zeros((HP, 8), jnp.float32).at[:hid, 0].set(W3[:, 0])
    BM = 2000
    assert n % BM == 0
    out = pl.pallas_call(
        _ffnn_body,
        grid=(n // BM,),
        in_specs=[
            pl.BlockSpec((BM, d_in), lambda i: (i, 0)),
            pl.BlockSpec((d_in, HP), lambda i: (0, 0)),
            pl.BlockSpec((8, HP), lambda i: (0, 0)),
            pl.BlockSpec((HP, HP), lambda i: (0, 0)),
            pl.BlockSpec((8, HP), lambda i: (0, 0)),
            pl.BlockSpec((HP, 8), lambda i: (0, 0)),
        ],
        out_specs=pl.BlockSpec((BM, 8), lambda i: (i, 0)),
        out_shape=jax.ShapeDtypeStruct((n, 8), jnp.float32),
    )(g_i, W1p, b1p, W2p, b2p, W3p)
    return out[:, 0]


def _make_sc_prune(n_pad, n_real):
    assert n_pad % (NT * L) == 0
    PT = n_pad // NT
    NG = PT // L
    NW = n_pad // 32
    WPT = NW // NT
    assert WPT % 8 == 0
    NGGLOB = n_pad // L
    iota = lambda: lax.iota(jnp.int32, L)

    mesh = plsc.VectorSubcoreMesh(
        core_axis_name="c", subcore_axis_name="s",
        num_cores=1, num_subcores=NT)

    @functools.partial(
        pl.kernel,
        out_type=[
            jax.ShapeDtypeStruct((n_pad,), jnp.float32),
            jax.ShapeDtypeStruct((n_pad,), jnp.int32),
            jax.ShapeDtypeStruct((NW,), jnp.int32),
            jax.ShapeDtypeStruct((16,), jnp.int32),
        ],
        mesh=mesh,
        compiler_params=pltpu.CompilerParams(needs_layout_passes=False),
        scratch_types=[
            pltpu.VMEM((n_pad,), jnp.int32),
            pltpu.VMEM((n_pad,), jnp.int32),
            pltpu.VMEM((n_pad,), jnp.int32),
            pltpu.VMEM((n_pad,), jnp.float32),
            pltpu.VMEM((n_pad,), jnp.int32),
            pltpu.VMEM((NW,), jnp.int32),
            pltpu.VMEM((NW,), jnp.int32),
            pltpu.VMEM((NW,), jnp.int32),
            pltpu.VMEM((NGGLOB,), jnp.int32),
            pltpu.VMEM((NGGLOB,), jnp.int32),
            pltpu.VMEM((16,), jnp.int32),
            pltpu.VMEM((16,), jnp.int32),
            pltpu.VMEM((8 * NT,), jnp.int32),
            pltpu.VMEM((256 * L,), jnp.int32),
            pltpu.SemaphoreType.DMA,
            pltpu.VMEM_SHARED((NW,), jnp.int32),
            pltpu.VMEM_SHARED((8 * NT,), jnp.int32),
            pltpu.VMEM_SHARED((n_pad,), jnp.int32),
            pltpu.VMEM_SHARED((n_pad,), jnp.int32),
        ],
    )
    def prune(sm_hbm, st_hbm, ln_hbm, k_hbm, bits_hbm,
              out_s_hbm, out_m_hbm, bits_out_hbm, flag_hbm,
              pk_v, aux_v, key_v, sm_v, wlim_v, bits_v, prev_v, mapfl_v,
              grng_v, dirty_v, kv_v, fb_v, fl_v, hist_v, stg_sem, sh_bits,
              sh_flags, sh_pk, sh_key):
        wid = lax.axis_index("s") + lax.axis_index("c") * NT

        c1 = pltpu.async_copy(sm_hbm, sm_v.at[pl.ds(0, n_real)], stg_sem)
        c2 = pltpu.async_copy(st_hbm, pk_v.at[pl.ds(0, n_real)], stg_sem)
        c3 = pltpu.async_copy(ln_hbm, aux_v.at[pl.ds(0, n_real)], stg_sem)
        c4 = pltpu.async_copy(k_hbm, kv_v, stg_sem)
        c5 = pltpu.async_copy(bits_hbm, bits_v, stg_sem)
        c1.wait()
        c2.wait()
        c3.wait()
        c4.wait()
        c5.wait()
        kth = kv_v[...][0]

        def rd_wvec(ref, wd):
            idx = wd + jnp.zeros((L,), jnp.int32)
            return plsc.load_gather(ref, [idx])

        def rd_word(ref, wd):
            return rd_wvec(ref, wd)[0]

        def init_g(g, _):
            gg = wid * NG + g
            sl = pl.ds(gg * L, L)
            ivec = gg * L + iota()
            valid = ivec < n_real
            st = jnp.where(valid, pk_v[sl], 50000 + ivec)
            ln = jnp.where(valid, aux_v[sl], 0)
            pk_v[sl] = (st << 4) | ln
            b = lax.bitcast_convert_type(sm_v[sl], jnp.int32)
            key = jnp.where(b >= 0, b, b ^ np.int32(0x7FFFFFFF))
            key_v[sl] = jnp.where(valid, key, INT_MIN)
            return 0

        lax.fori_loop(0, NG, init_g, 0)
        pltpu.sync_copy(pk_v.at[pl.ds(wid * PT, PT)],
                        sh_pk.at[pl.ds(wid * PT, PT)])
        pltpu.sync_copy(key_v.at[pl.ds(wid * PT, PT)],
                        sh_key.at[pl.ds(wid * PT, PT)])
        plsc.subcore_barrier()
        pltpu.sync_copy(sh_pk, pk_v)
        pltpu.sync_copy(sh_key, key_v)
        plsc.subcore_barrier()

        def srch(g, _):
            base = wid * PT + g * L
            ivec = base + iota()
            sl = pl.ds(base, L)
            pk_i = pk_v[sl]
            s_i = jnp.right_shift(pk_i, 4)
            e_i = s_i + (pk_i & 15)
            t_lo = s_i - 9

            base_lo = jnp.maximum(ivec - 128, 0)
            ub = jnp.minimum(ivec + 128, n_pad)
            pl_probe = jnp.right_shift(
                plsc.load_gather(pk_v, [base_lo]), 4)
            pr_probe = jnp.right_shift(
                plsc.load_gather(pk_v, [jnp.minimum(ub, n_pad - 1)]), 4)
            ovf_l = (ivec > 128) & (pl_probe >= t_lo)
            ovf_r = (ivec + 128 < n_pad) & (pr_probe <= e_i)

            def bstep(p, pos, leq, tgt, lim):
                step = jnp.right_shift(np.int32(128), p)
                cand = pos + step
                jg = jnp.clip(cand - 1, 0, n_pad - 1)
                s_c = jnp.right_shift(plsc.load_gather(pk_v, [jg]), 4)
                less = (s_c <= tgt) if leq else (s_c < tgt)
                ok = (cand <= lim) & less
                return jnp.where(ok, cand, pos)

            def lh_loop(p, st):
                plo, phi = st
                return (bstep(p, plo, False, t_lo, ivec),
                        bstep(p, phi, True, e_i, ub))

            lo, hi = lax.fori_loop(0, 8, lh_loop, (base_lo, ivec), unroll=4)
            dl = jnp.where(ovf_l, ivec, ivec - lo)
            dr = jnp.where(ovf_r, n_pad - 1 - ivec, hi - ivec - 1)
            wlim_v[sl] = (dl << 16) | dr
            lo_w = jnp.clip(jnp.right_shift(base - jnp.max(dl), 5), 0, NW - 1)
            hi_w = jnp.clip(jnp.right_shift(base + 15 + jnp.max(dr), 5),
                            0, NW - 1)
            gg = wid * NG + g
            wb = (gg >> 4) << 4
            lane = gg - wb
            blk = grng_v[pl.ds(wb, L)]
            grng_v[pl.ds(wb, L)] = jnp.where(
                iota() == lane, (lo_w << 16) | hi_w, blk)
            return 0

        lax.fori_loop(0, NG, srch, 0, unroll=2)

        def init_pd(mw, _):
            sl = pl.ds(mw * L, L)
            prev_v[sl] = bits_v[sl]
            return 0

        lax.fori_loop(0, NW // L, init_pd, 0)

        def init_d(gi, _):
            dirty_v[pl.ds(wid * NG + gi * L, L)] = jnp.full((L,), 1, jnp.int32)
            return 0

        lax.fori_loop(0, NG // L, init_d, 0)

        def kp_bits(jc):
            w = plsc.load_gather(bits_v, [jnp.right_shift(jc, 5)])
            return jnp.right_shift(w, jc & 31) & 1

        def sweep(s, prev):
            fb_v[...] = jnp.zeros((L,), jnp.int32)

            @pl.when(prev != 0)
            def _do_sweep():
                def group(g, _):
                    gg = wid * NG + g
                    dirt = rd_word(dirty_v, gg)

                    @pl.when(dirt != 0)
                    def _scan():
                        base = wid * PT + g * L
                        ivec = base + iota()
                        sl = pl.ds(base, L)
                        pk_i = pk_v[sl]
                        s_i = jnp.right_shift(pk_i, 4)
                        e_i = s_i + (pk_i & 15)
                        key_i = key_v[sl]
                        wl = wlim_v[sl]
                        dl = jnp.right_shift(wl, 16)
                        dr = wl & 65535
                        val_i = ivec < n_real

                        def win_body(d, thr):
                            jl = jnp.clip(ivec - d, 0, n_pad - 1)
                            jr = jnp.clip(ivec + d, 0, n_pad - 1)
                            pk_l = plsc.load_gather(pk_v, [jl])
                            pk_r = plsc.load_gather(pk_v, [jr])
                            key_l = plsc.load_gather(key_v, [jl])
                            key_r = plsc.load_gather(key_v, [jr])
                            kp_l = kp_bits(jl)
                            kp_r = kp_bits(jr)
                            s_l = jnp.right_shift(pk_l, 4)
                            e_l = s_l + (pk_l & 15)
                            s_r = jnp.right_shift(pk_r, 4)
                            e_r = s_r + (pk_r & 15)
                            hit = (d <= dl) & (s_l < s_i) & (s_i <= e_l) & \
                                (e_l < e_i) & (key_l >= key_i) & (kp_l == 1)
                            hit = hit | ((d <= dr) & (s_r > s_i) &
                                         (e_r > e_i) & (key_r > key_i) &
                                         (kp_r == 1))
                            return thr | jnp.where(hit, 1, 0)

                        z16 = jnp.zeros((L,), jnp.int32)
                        dmax = jnp.maximum(jnp.max(dl), jnp.max(dr))
                        thr = plsc.parallel_loop(
                            np.int32(1), dmax + 1, unroll=2,
                            carry=z16)(win_body)
                        new_keep = jnp.where((thr == 0) & val_i, 1, 0)
                        hw = jnp.sum(new_keep << iota())
                        wd = jnp.right_shift(gg, 1)
                        sh = (gg & 1) * 16
                        wb = (wd >> 4) << 4
                        lane = wd - wb
                        blk = bits_v[pl.ds(wb, L)]
                        old = rd_wvec(bits_v, wd)[0]
                        neww = (old & ~(65535 << sh)) | (hw << sh)
                        bits_v[pl.ds(wb, L)] = jnp.where(
                            iota() == lane, neww, blk)
                        ch = jnp.where(neww != old, 1, 0)
                        fb_v[...] = fb_v[...] | jnp.full((L,), ch, jnp.int32)

                    return 0

                lax.fori_loop(0, NG, group, 0)

            pltpu.sync_copy(bits_v.at[pl.ds(wid * WPT, WPT)],
                            sh_bits.at[pl.ds(wid * WPT, WPT)])
            pltpu.sync_copy(fb_v.at[pl.ds(0, 8)],
                            sh_flags.at[pl.ds(wid * 8, 8)])
            plsc.subcore_barrier()
            pltpu.sync_copy(sh_bits, bits_v)
            pltpu.sync_copy(sh_flags, fl_v)
            plsc.subcore_barrier()

            @pl.when(prev != 0)
            def _mark_dirty():
                def bld(mw, _):
                    msl = pl.ds(mw * L, L)
                    nv = bits_v[msl]
                    mapfl_v[msl] = jnp.where(nv != prev_v[msl], 1, 0)
                    prev_v[msl] = nv
                    return 0

                lax.fori_loop(0, NW // L, bld, 0)

                def mkd(gi, _):
                    gsl = pl.ds(wid * NG + gi * L, L)
                    rng = grng_v[gsl]
                    lo_w = jnp.right_shift(rng, 16)
                    span = (rng & 65535) - lo_w
                    d = jnp.where(span > 7, 1, 0)
                    for t in range(8):
                        f = plsc.load_gather(
                            mapfl_v, [jnp.clip(lo_w + t, 0, NW - 1)])
                        d = d | jnp.where((t <= span) & (f == 1), 1, 0)
                    dirty_v[gsl] = d
                    return 0

                lax.fori_loop(0, NG // L, mkd, 0)

            def orf(t, a):
                return a | fl_v[pl.ds(t * L, L)]

            vacc = lax.fori_loop(0, 8 * NT // L, orf,
                                 jnp.zeros((L,), jnp.int32))
            return jnp.any(vacc != 0).astype(jnp.int32)

        not_conv = lax.fori_loop(0, S_INNER, sweep, np.int32(1))

        pltpu.sync_copy(bits_v.at[pl.ds(wid * WPT, WPT)],
                        bits_out_hbm.at[pl.ds(wid * WPT, WPT)])
        fb_v[...] = jnp.full((L,), not_conv, jnp.int32)

        @pl.when(wid == 0)
        def _wflag():
            pltpu.sync_copy(fb_v, flag_hbm)

        def unpack16(gg):
            wd = jnp.right_shift(gg, 1)
            sh = (gg & 1) * 16
            wv = rd_wvec(bits_v, wd)
            return jnp.right_shift(wv, sh + iota()) & 1

        def cnt_g(g, acc):
            kp = unpack16(wid * NG + g)
            return acc + jnp.sum(kp)

        my_cnt = lax.fori_loop(0, NG, cnt_g, np.int32(0))
        fb_v[...] = jnp.full((L,), my_cnt, jnp.int32)
        pltpu.sync_copy(fb_v.at[pl.ds(0, 8)], sh_flags.at[pl.ds(wid * 8, 8)])
        plsc.subcore_barrier()
        pltpu.sync_copy(sh_flags, fl_v)
        plsc.subcore_barrier()

        def sumf(t, a):
            return a + fl_v[pl.ds(t * L, L)]

        vsum = lax.fori_loop(0, 8 * NT // L, sumf, jnp.zeros((L,), jnp.int32))
        total = jnp.right_shift(jnp.sum(vsum), 3)

        fb_v[...] = jnp.where(iota() < 8, INT_MIN, 0)

        @pl.when(total > kth)
        def _select():
            def level(p, carry):
                rem, hi = carry
                shift = 24 - 8 * p

                def zero_h(w, _):
                    hist_v[pl.ds(w * L, L)] = jnp.zeros((L,), jnp.int32)
                    return 0

                lax.fori_loop(0, 256, zero_h, 0)

                def acc_g(g, _):
                    sl = pl.ds(g * L, L)
                    key = key_v[sl]
                    kp = unpack16(g)
                    pref_ok = jnp.where(
                        p == 0,
                        jnp.ones((L,), jnp.bool_),
                        (key >> jnp.minimum(shift + 8, 31)) == hi)
                    cand = jnp.where((kp == 1) & pref_ok, 1, 0)
                    bn = jnp.where(p == 0, (key >> 24) + 128,
                                   (key >> shift) & 255)
                    plsc.addupdate_scatter(hist_v, [bn * L + iota()], cand)
                    return 0

                lax.fori_loop(0, NGGLOB, acc_g, 0)

                def scan_b(br, st):
                    b = 255 - br
                    found, bstar, acc, rem_n = st
                    hb = jnp.sum(hist_v[pl.ds(b * L, L)])
                    acc2 = acc + hb
                    take = (found == 0) & (acc2 >= rem)
                    bstar = jnp.where(take, b, bstar)
                    rem_n = jnp.where(take, rem - acc, rem_n)
                    found = jnp.where(take, 1, found)
                    return (found, bstar, acc2, rem_n)

                _, bstar, _, rem_n = lax.fori_loop(
                    0, 256, scan_b,
                    (np.int32(0), np.int32(0), np.int32(0), rem))
                bval = jnp.where(p == 0, bstar - 128, bstar)
                return (rem_n, (hi << 8) | bval)

            rem, hi = lax.fori_loop(0, 4, level, (kth, np.int32(0)))
            vstar = hi
            fb_v[...] = jnp.where(iota() < 8, vstar, rem)

            def rank_g(g, c):
                sl = pl.ds(g * L, L)
                key = key_v[sl]
                kp = unpack16(g)
                tie = jnp.where((kp == 1) & (key == vstar), 1, 0)
                incl = jnp.cumsum(tie)
                pk_v[sl] = c + incl - tie
                return c + jnp.sum(tie)

            lax.fori_loop(0, NGGLOB, rank_g, np.int32(0))

        fbv = fb_v[...]
        vstar = fbv[0]
        mquota = fbv[8]

        def out_g(g, _):
            gg = wid * NG + g
            base = gg * L
            sl = pl.ds(base, L)
            kp = unpack16(gg)
            key = key_v[sl]
            rank = pk_v[sl]
            fin = (kp == 1) & ((key > vstar) |
                               ((key == vstar) & (rank < mquota)))
            aux_v[sl] = jnp.where(fin, 1, 0)
            sm_v[sl] = jnp.where(fin, sm_v[sl], 0.0)
            return 0

        lax.fori_loop(0, NG, out_g, 0)
        pltpu.sync_copy(sm_v.at[pl.ds(wid * PT, PT)],
                        out_s_hbm.at[pl.ds(wid * PT, PT)])
        pltpu.sync_copy(aux_v.at[pl.ds(wid * PT, PT)],
                        out_m_hbm.at[pl.ds(wid * PT, PT)])

    return prune


def kernel(g_i, W1, b1, W2, b2, W3, b3, span_start, span_len, T):
    n = g_i.shape[0]
    s_m = _ffnn_scores(g_i, W1, b1, W2, b2, W3) + b3[0]

    n_pad = ((n + 4095) // 4096) * 4096
    st32 = span_start.astype(jnp.int32)
    ln32 = span_len.astype(jnp.int32)
    k = (0.4 * jnp.asarray(T).astype(jnp.float32)).astype(jnp.int32)
    k_arr = jnp.full((16,), k, jnp.int32)

    nw = n_pad // 32
    widx = jnp.arange(nw, dtype=jnp.int32)
    w_last = n // 32
    rem_bits = n % 32
    last_val = (1 << rem_bits) - 1 if rem_bits else 0
    bits0 = jnp.where(widx < w_last, np.int32(-1), np.int32(0))
    bits0 = jnp.where(widx == w_last, np.int32(last_val), bits0)

    prune = _make_sc_prune(n_pad, n)

    def cond(carry):
        return carry[3] != 0

    def body(carry):
        bits, _, _, _ = carry
        out_s, out_m, bits2, flag = prune(s_m, st32, ln32, k_arr, bits)
        return (bits2, out_s, out_m, flag[0])

    _, out_s, out_m, _ = lax.while_loop(
        cond, body,
        (bits0, jnp.zeros((n_pad,), jnp.float32),
         jnp.zeros((n_pad,), jnp.int32), np.int32(1)))
    pruned = out_s[:n]
    mask = out_m[:n].astype(bool)
    return pruned, mask
